# Initial kernel scaffold; baseline (speedup 1.0000x reference)
import jax
import jax.numpy as jnp
from jax.experimental import pallas as pl


def kernel(x_pfc, x_vtx, batch_pfc, batch_vtx, pfc_w1, pfc_b1, pfc_w2, pfc_b2, vtx_w1, vtx_b1, vtx_w2, vtx_b2, conv_w, conv_b, out_w1, out_b1, out_w2, out_b2, out_w3, out_b3, out_w4, out_b4):
    raise NotImplementedError("write your pallas kernel here")


# speedup vs baseline: 1.0152x; 1.0152x over previous
import jax
import jax.numpy as jnp
from jax.experimental import pallas as pl


def _lrelu(x):
    return jax.nn.leaky_relu(x, 0.01)


def _mlp_body(h_ref, w1_ref, b1_ref, w2_ref, b2_ref, w3_ref, b3_ref, w4_ref, b4_ref, o_ref):
    h = h_ref[...]
    h = _lrelu(jnp.dot(h, w1_ref[...], preferred_element_type=jnp.float32) + b1_ref[...])
    h = _lrelu(jnp.dot(h, w2_ref[...], preferred_element_type=jnp.float32) + b2_ref[...])
    h = _lrelu(jnp.dot(h, w3_ref[...], preferred_element_type=jnp.float32) + b3_ref[...])
    h = _lrelu(jnp.dot(h, w4_ref[...], preferred_element_type=jnp.float32) + b4_ref[...])
    o_ref[...] = h


def _out_mlp(feats2, out_w1, out_b1, out_w2, out_b2, out_w3, out_b3, out_w4, out_b4):
    n = feats2.shape[0]
    w3p = jnp.zeros((32, 128), jnp.float32).at[:, :4].set(out_w3)
    b3p = jnp.zeros((1, 128), jnp.float32).at[0, :4].set(out_b3)
    w4p = jnp.zeros((128, 128), jnp.float32).at[:4, :1].set(out_w4)
    b4p = jnp.zeros((1, 128), jnp.float32).at[0, :1].set(out_b4)

    def const2(a):
        return pl.BlockSpec(a.shape, lambda i: (0, 0))

    b1 = out_b1.reshape(1, -1)
    b2 = out_b2.reshape(1, -1)
    out = pl.pallas_call(
        _mlp_body,
        out_shape=jax.ShapeDtypeStruct((n, 128), jnp.float32),
        grid=(n // 2000,),
        in_specs=[pl.BlockSpec((2000, 32), lambda i: (i, 0)),
                  const2(out_w1), const2(b1), const2(out_w2), const2(b2),
                  const2(w3p), const2(b3p), const2(w4p), const2(b4p)],
        out_specs=pl.BlockSpec((2000, 128), lambda i: (i, 0)),
    )(feats2, out_w1, b1, out_w2, b2, w3p, b3p, w4p, b4p)
    return out[:, :1]


def _knn_idx(x_src, x_dst, k):
    d = (jnp.sum(x_dst ** 2, axis=1)[:, None]
         + jnp.sum(x_src ** 2, axis=1)[None, :]
         - 2.0 * (x_dst @ x_src.T))
    _, idx = jax.lax.top_k(-d, k)
    return idx


def _edge_conv(x_src, x_dst, idx, W, b):
    d = x_src.shape[1]
    Wt, Wb = W[:d], W[d:]
    u = x_dst @ (Wt - Wb) + b
    v = x_src @ Wb
    vmax = jnp.max(v[idx], axis=1)
    return _lrelu(u + vmax)


def kernel(x_pfc, x_vtx, batch_pfc, batch_vtx, pfc_w1, pfc_b1, pfc_w2, pfc_b2,
           vtx_w1, vtx_b1, vtx_w2, vtx_b2, conv_w, conv_b,
           out_w1, out_b1, out_w2, out_b2, out_w3, out_b3, out_w4, out_b4):
    k = 16
    x_pfc_enc = _lrelu(_lrelu(x_pfc @ pfc_w1 + pfc_b1) @ pfc_w2 + pfc_b2)
    x_vtx_enc = _lrelu(_lrelu(x_vtx @ vtx_w1 + vtx_b1) @ vtx_w2 + vtx_b2)
    idx1 = _knn_idx(x_pfc_enc, x_pfc_enc, k)
    feats1 = _edge_conv(x_pfc_enc, x_pfc_enc, idx1, conv_w, conv_b)
    idx2 = _knn_idx(x_vtx_enc, feats1, k)
    feats2 = _edge_conv(x_vtx_enc, feats1, idx2, conv_w, conv_b)
    out = _out_mlp(feats2, out_w1, out_b1, out_w2, out_b2, out_w3, out_b3, out_w4, out_b4)
    return (out, batch_pfc)


# speedup vs baseline: 1.1937x; 1.1758x over previous
import functools

import jax
import jax.numpy as jnp
from jax import lax
from jax.experimental import pallas as pl
from jax.experimental.pallas import tpu as pltpu
from jax.experimental.pallas import tpu_sc as plsc

N_PFC = 10000
N_VTX = 1000
NP_PAD = 10240
NV_PAD = 1024
AUG = 40
NEG = -3.0e38


def _lrelu(x):
    return jax.nn.leaky_relu(x, 0.01)


def _pfc_prep_body(x_ref, w1_ref, b1_ref, w2_ref, b2_ref, wd_ref, cb_ref, wb_ref,
                   xd_ref, xs_ref, u_ref, v_ref):
    i = pl.program_id(0)
    x = x_ref[...]
    enc = _lrelu(jnp.dot(x, w1_ref[...], preferred_element_type=jnp.float32) + b1_ref[...])
    enc = _lrelu(jnp.dot(enc, w2_ref[...], preferred_element_type=jnp.float32) + b2_ref[...])
    n = enc.shape[0]
    rows = i * n + lax.broadcasted_iota(jnp.int32, (n, 1), 0)
    valid = rows < N_PFC
    n2 = jnp.sum(enc * enc, axis=1, keepdims=True)
    n2t = jnp.where(valid, -n2, NEG)
    ones = jnp.ones((n, 1), jnp.float32)
    zer = jnp.zeros((n, AUG - 33), jnp.float32)
    xd_ref[...] = jnp.concatenate([enc, ones, zer], axis=1)
    xs_ref[...] = jnp.concatenate([2.0 * enc, n2t, zer], axis=1)
    u_ref[...] = jnp.dot(enc, wd_ref[...], preferred_element_type=jnp.float32) + cb_ref[...]
    v = jnp.dot(enc, wb_ref[...], preferred_element_type=jnp.float32)
    v_ref[...] = jnp.concatenate([v, jnp.zeros((n, 96), jnp.float32)], axis=1)


def _pfc_prep(xp, w1p, b1, w2, b2, wd, cb, wb):
    blk = 2048
    grid = (NP_PAD // blk,)
    c = lambda a: pl.BlockSpec(a.shape, lambda i: (0,) * a.ndim)
    return pl.pallas_call(
        _pfc_prep_body,
        grid=grid,
        in_specs=[pl.BlockSpec((blk, 8), lambda i: (i, 0)),
                  c(w1p), c(b1), c(w2), c(b2), c(wd), c(cb), c(wb)],
        out_specs=[pl.BlockSpec((blk, AUG), lambda i: (i, 0)),
                   pl.BlockSpec((blk, AUG), lambda i: (i, 0)),
                   pl.BlockSpec((blk, 32), lambda i: (i, 0)),
                   pl.BlockSpec((blk, 128), lambda i: (i, 0))],
        out_shape=[jax.ShapeDtypeStruct((NP_PAD, AUG), jnp.float32),
                   jax.ShapeDtypeStruct((NP_PAD, AUG), jnp.float32),
                   jax.ShapeDtypeStruct((NP_PAD, 32), jnp.float32),
                   jax.ShapeDtypeStruct((NP_PAD, 128), jnp.float32)],
    )(xp, w1p, b1, w2, b2, wd, cb, wb)


def _vtx_prep_body(x_ref, w1_ref, b1_ref, w2_ref, b2_ref, wb_ref, xs_ref, v_ref):
    x = x_ref[...]
    enc = _lrelu(jnp.dot(x, w1_ref[...], preferred_element_type=jnp.float32) + b1_ref[...])
    enc = _lrelu(jnp.dot(enc, w2_ref[...], preferred_element_type=jnp.float32) + b2_ref[...])
    n = enc.shape[0]
    rows = lax.broadcasted_iota(jnp.int32, (n, 1), 0)
    valid = rows < N_VTX
    n2 = jnp.sum(enc * enc, axis=1, keepdims=True)
    n2t = jnp.where(valid, -n2, NEG)
    zer = jnp.zeros((n, AUG - 33), jnp.float32)
    xs_ref[...] = jnp.concatenate([2.0 * enc, n2t, zer], axis=1)
    v = jnp.dot(enc, wb_ref[...], preferred_element_type=jnp.float32)
    v_ref[...] = jnp.concatenate([v, jnp.zeros((n, 96), jnp.float32)], axis=1)


def _vtx_prep(xv, w1p, b1, w2, b2, wb):
    c = lambda a: pl.BlockSpec(a.shape, lambda: (0,) * a.ndim)
    return pl.pallas_call(
        _vtx_prep_body,
        in_specs=[c(xv), c(w1p), c(b1), c(w2), c(b2), c(wb)],
        out_specs=[c(jnp.zeros((NV_PAD, AUG))), c(jnp.zeros((NV_PAD, 128)))],
        out_shape=[jax.ShapeDtypeStruct((NV_PAD, AUG), jnp.float32),
                   jax.ShapeDtypeStruct((NV_PAD, 128), jnp.float32)],
    )(xv, w1p, b1, w2, b2, wb)


def _score_body(xd_ref, xs_ref, s_ref):
    s_ref[...] = lax.dot_general(
        xd_ref[...], xs_ref[...], (((1,), (1,)), ((), ())),
        preferred_element_type=jnp.float32)


def _scores(xd_aug, xs_aug, src_blk):
    nd, ns = xd_aug.shape[0], xs_aug.shape[0]
    dst_blk = 256
    grid = (nd // dst_blk, ns // src_blk)
    return pl.pallas_call(
        _score_body,
        grid=grid,
        in_specs=[pl.BlockSpec((dst_blk, AUG), lambda i, j: (i, 0)),
                  pl.BlockSpec((src_blk, AUG), lambda i, j: (j, 0))],
        out_specs=pl.BlockSpec((dst_blk, src_blk), lambda i, j: (i, j)),
        out_shape=jax.ShapeDtypeStruct((nd, ns), jnp.float32),
    )(xd_aug, xs_aug)


def _make_topk_gather(ncols, rows_per_blk):
    nvregs = ncols // 16
    rpt = NP_PAD // 32
    nblk = rpt // rows_per_blk
    assert nblk % 2 == 0
    nb_gather = rpt // 8
    mesh = plsc.VectorSubcoreMesh(core_axis_name="c", subcore_axis_name="s")

    @functools.partial(
        pl.kernel,
        out_type=jax.ShapeDtypeStruct((NP_PAD * 32,), jnp.float32),
        mesh=mesh,
        compiler_params=pltpu.CompilerParams(needs_layout_passes=False),
        scratch_types=[
            pltpu.VMEM((rows_per_blk, ncols), jnp.float32),
            pltpu.VMEM((rows_per_blk, ncols), jnp.float32),
            pltpu.VMEM((nb_gather, 128), jnp.int32),
            pltpu.VMEM((128, 128), jnp.float32),
            pltpu.VMEM((rpt * 32,), jnp.float32),
            pltpu.SemaphoreType.DMA,
            pltpu.SemaphoreType.DMA,
            pltpu.SemaphoreType.DMA,
        ],
    )
    def topk_kernel(s_hbm, v_hbm, out_hbm, rb0, rb1, idxb, gb, ob, sem0, sem1, gsem):
        cid = lax.axis_index("c")
        sid = lax.axis_index("s")
        wid = sid * 2 + cid
        row_base = wid * rpt
        iota16 = lax.iota(jnp.int32, 16)
        neg16 = jnp.full((16,), NEG, jnp.float32)

        def scan_row(rbuf, r, row_local):
            def vbody(t, car):
                tk, tv, tmin = car
                vals = rbuf[r, pl.ds(t * 16, 16)]
                cnt = plsc.all_reduce_population_count(vals > tmin)
                hit = cnt[0] > 0

                def do_merge(op):
                    tk0, tv0 = op
                    ids = t * 16 + iota16
                    ksk, ksv = plsc.sort_key_val(vals, ids, descending=False)
                    sel = tk0 >= ksk
                    nk = jnp.where(sel, tk0, ksk)
                    ni = jnp.where(sel, tv0, ksv)
                    ntk, ntv = plsc.sort_key_val(nk, ni, descending=True)
                    return ntk, ntv, ntk[15]

                return lax.cond(hit, do_merge, lambda op: (op[0], op[1], tmin),
                                (tk, tv))

            tk, tv, _ = lax.fori_loop(
                0, nvregs, vbody, (neg16, neg16, jnp.float32(NEG)))
            b_idx = row_local // 8
            lane = (row_local % 8) * 16
            idxb[b_idx, pl.ds(lane, 16)] = tv.astype(jnp.int32)

        def process_blk(rbuf, blk):
            for r in range(rows_per_blk):
                scan_row(rbuf, r, blk * rows_per_blk + r)

        pltpu.async_copy(s_hbm.at[pl.ds(row_base, rows_per_blk)], rb0, sem0)

        def pair_body(p, _):
            b0 = p * 2
            off1 = row_base + (b0 + 1) * rows_per_blk
            pltpu.async_copy(s_hbm.at[pl.ds(off1, rows_per_blk)], rb1, sem1)
            pltpu.make_async_copy(s_hbm.at[pl.ds(row_base, rows_per_blk)], rb0, sem0).wait()
            process_blk(rb0, b0)
            off2 = jnp.minimum(row_base + (b0 + 2) * rows_per_blk,
                               NP_PAD - rows_per_blk)
            pltpu.async_copy(s_hbm.at[pl.ds(off2, rows_per_blk)], rb0, sem0)
            pltpu.make_async_copy(s_hbm.at[pl.ds(off1, rows_per_blk)], rb1, sem1).wait()
            process_blk(rb1, b0 + 1)
            return 0

        lax.fori_loop(0, nblk // 2, pair_body, 0)
        pltpu.make_async_copy(s_hbm.at[pl.ds(row_base, rows_per_blk)], rb0, sem0).wait()

        def gather_body(b, _):
            pltpu.async_copy(v_hbm.at[idxb.at[b]], gb, gsem).wait()
            for rr in range(8):
                m0 = gb[rr * 16, pl.ds(0, 16)]
                m1 = gb[rr * 16, pl.ds(16, 16)]
                for j in range(1, 16):
                    m0 = jnp.maximum(m0, gb[rr * 16 + j, pl.ds(0, 16)])
                    m1 = jnp.maximum(m1, gb[rr * 16 + j, pl.ds(16, 16)])
                row = b * 8 + rr
                ob[pl.ds(row * 32, 16)] = m0
                ob[pl.ds(row * 32 + 16, 16)] = m1
            return 0

        lax.fori_loop(0, nb_gather, gather_body, 0)
        pltpu.sync_copy(ob, out_hbm.at[pl.ds(row_base * 32, rpt * 32)])

    return topk_kernel


_topk_cache = {}


def _topk_gather_1(s, v):
    if 1 not in _topk_cache:
        _topk_cache[1] = _make_topk_gather(NP_PAD, 4)
    return _topk_cache[1](s, v)


def _topk_gather_2(s, v):
    if 2 not in _topk_cache:
        _topk_cache[2] = _make_topk_gather(NV_PAD, 16)
    return _topk_cache[2](s, v)


def _post1_body(u_ref, mv_ref, wd_ref, cb_ref, xd_ref, u2_ref):
    f = _lrelu(u_ref[...] + mv_ref[...])
    n = f.shape[0]
    ones = jnp.ones((n, 1), jnp.float32)
    zer = jnp.zeros((n, AUG - 33), jnp.float32)
    xd_ref[...] = jnp.concatenate([f, ones, zer], axis=1)
    u2_ref[...] = jnp.dot(f, wd_ref[...], preferred_element_type=jnp.float32) + cb_ref[...]


def _post1(u1, mv1, wd, cb):
    blk = 2048
    c = lambda a: pl.BlockSpec(a.shape, lambda i: (0,) * a.ndim)
    return pl.pallas_call(
        _post1_body,
        grid=(NP_PAD // blk,),
        in_specs=[pl.BlockSpec((blk, 32), lambda i: (i, 0)),
                  pl.BlockSpec((blk, 32), lambda i: (i, 0)), c(wd), c(cb)],
        out_specs=[pl.BlockSpec((blk, AUG), lambda i: (i, 0)),
                   pl.BlockSpec((blk, 32), lambda i: (i, 0))],
        out_shape=[jax.ShapeDtypeStruct((NP_PAD, AUG), jnp.float32),
                   jax.ShapeDtypeStruct((NP_PAD, 32), jnp.float32)],
    )(u1, mv1, wd, cb)


def _mlp_body(u_ref, mv_ref, w1_ref, b1_ref, w2_ref, b2_ref, w3_ref, b3_ref,
              w4_ref, b4_ref, o_ref):
    h = _lrelu(u_ref[...] + mv_ref[...])
    h = _lrelu(jnp.dot(h, w1_ref[...], preferred_element_type=jnp.float32) + b1_ref[...])
    h = _lrelu(jnp.dot(h, w2_ref[...], preferred_element_type=jnp.float32) + b2_ref[...])
    h = _lrelu(jnp.dot(h, w3_ref[...], preferred_element_type=jnp.float32) + b3_ref[...])
    h = _lrelu(jnp.dot(h, w4_ref[...], preferred_element_type=jnp.float32) + b4_ref[...])
    o_ref[...] = h


def _out_mlp(u2, mv2, out_w1, out_b1, out_w2, out_b2, out_w3, out_b3, out_w4, out_b4):
    blk = 2048
    w3p = jnp.zeros((32, 128), jnp.float32).at[:, :4].set(out_w3)
    b3p = jnp.zeros((1, 128), jnp.float32).at[0, :4].set(out_b3)
    w4p = jnp.zeros((128, 128), jnp.float32).at[:4, :1].set(out_w4)
    b4p = jnp.zeros((1, 128), jnp.float32).at[0, :1].set(out_b4)
    b1 = out_b1.reshape(1, -1)
    b2 = out_b2.reshape(1, -1)
    c = lambda a: pl.BlockSpec(a.shape, lambda i: (0, 0))
    out = pl.pallas_call(
        _mlp_body,
        grid=(NP_PAD // blk,),
        in_specs=[pl.BlockSpec((blk, 32), lambda i: (i, 0)),
                  pl.BlockSpec((blk, 32), lambda i: (i, 0)),
                  c(out_w1), c(b1), c(out_w2), c(b2),
                  c(w3p), c(b3p), c(w4p), c(b4p)],
        out_specs=pl.BlockSpec((blk, 128), lambda i: (i, 0)),
        out_shape=jax.ShapeDtypeStruct((NP_PAD, 128), jnp.float32),
    )(u2, mv2, out_w1, b1, out_w2, b2, w3p, b3p, w4p, b4p)
    return out[:N_PFC, :1]


def kernel(x_pfc, x_vtx, batch_pfc, batch_vtx, pfc_w1, pfc_b1, pfc_w2, pfc_b2,
           vtx_w1, vtx_b1, vtx_w2, vtx_b2, conv_w, conv_b,
           out_w1, out_b1, out_w2, out_b2, out_w3, out_b3, out_w4, out_b4):
    wb = conv_w[32:]
    wd = conv_w[:32] - wb
    cb = conv_b.reshape(1, -1)

    xp = jnp.zeros((NP_PAD, 8), jnp.float32).at[:N_PFC, :7].set(x_pfc)
    w1p = jnp.zeros((8, 32), jnp.float32).at[:7].set(pfc_w1)
    xd1, xs1, u1, v1 = _pfc_prep(xp, w1p, pfc_b1.reshape(1, -1),
                                 pfc_w2, pfc_b2.reshape(1, -1), wd, cb, wb)

    xv = jnp.zeros((NV_PAD, 8), jnp.float32).at[:N_VTX, :4].set(x_vtx)
    vw1p = jnp.zeros((8, 32), jnp.float32).at[:4].set(vtx_w1)
    xs2, v2 = _vtx_prep(xv, vw1p, vtx_b1.reshape(1, -1), vtx_w2,
                        vtx_b2.reshape(1, -1), wb)

    s1 = _scores(xd1, xs1, 2048)
    mv1 = _topk_gather_1(s1, v1).reshape(NP_PAD, 32)
    xd2, u2 = _post1(u1, mv1, wd, cb)
    s2 = _scores(xd2, xs2, 1024)
    mv2 = _topk_gather_2(s2, v2).reshape(NP_PAD, 32)
    out = _out_mlp(u2, mv2, out_w1, out_b1, out_w2, out_b2,
                   out_w3, out_b3, out_w4, out_b4)
    return (out, batch_pfc)


# speedup vs baseline: 8.7889x; 7.3626x over previous
import functools

import jax
import jax.numpy as jnp
from jax import lax
from jax.experimental import pallas as pl
from jax.experimental.pallas import tpu as pltpu
from jax.experimental.pallas import tpu_sc as plsc

N_PFC = 10000
N_VTX = 1000
NP_PAD = 10240
NV_PAD = 1024
AUG = 40
NEG = -3.0e38


def _lrelu(x):
    return jax.nn.leaky_relu(x, 0.01)


def _pfc_prep_body(x_ref, w1_ref, b1_ref, w2_ref, b2_ref, wd_ref, cb_ref, wb_ref,
                   xd_ref, xs_ref, u_ref, v_ref):
    i = pl.program_id(0)
    x = x_ref[...]
    enc = _lrelu(jnp.dot(x, w1_ref[...], preferred_element_type=jnp.float32) + b1_ref[...])
    enc = _lrelu(jnp.dot(enc, w2_ref[...], preferred_element_type=jnp.float32) + b2_ref[...])
    n = enc.shape[0]
    rows = i * n + lax.broadcasted_iota(jnp.int32, (n, 1), 0)
    valid = rows < N_PFC
    n2 = jnp.sum(enc * enc, axis=1, keepdims=True)
    n2t = jnp.where(valid, -n2, NEG)
    ones = jnp.ones((n, 1), jnp.float32)
    zer = jnp.zeros((n, AUG - 33), jnp.float32)
    xd_ref[...] = jnp.concatenate([enc, ones, zer], axis=1)
    xs_ref[...] = jnp.concatenate([2.0 * enc, n2t, zer], axis=1)
    u_ref[...] = jnp.dot(enc, wd_ref[...], preferred_element_type=jnp.float32) + cb_ref[...]
    v = jnp.dot(enc, wb_ref[...], preferred_element_type=jnp.float32)
    v_ref[...] = jnp.concatenate([v, jnp.zeros((n, 96), jnp.float32)], axis=1)


def _pfc_prep(xp, w1p, b1, w2, b2, wd, cb, wb):
    blk = 2048
    grid = (NP_PAD // blk,)
    c = lambda a: pl.BlockSpec(a.shape, lambda i: (0,) * a.ndim)
    return pl.pallas_call(
        _pfc_prep_body,
        grid=grid,
        in_specs=[pl.BlockSpec((blk, 8), lambda i: (i, 0)),
                  c(w1p), c(b1), c(w2), c(b2), c(wd), c(cb), c(wb)],
        out_specs=[pl.BlockSpec((blk, AUG), lambda i: (i, 0)),
                   pl.BlockSpec((blk, AUG), lambda i: (i, 0)),
                   pl.BlockSpec((blk, 32), lambda i: (i, 0)),
                   pl.BlockSpec((blk, 128), lambda i: (i, 0))],
        out_shape=[jax.ShapeDtypeStruct((NP_PAD, AUG), jnp.float32),
                   jax.ShapeDtypeStruct((NP_PAD, AUG), jnp.float32),
                   jax.ShapeDtypeStruct((NP_PAD, 32), jnp.float32),
                   jax.ShapeDtypeStruct((NP_PAD, 128), jnp.float32)],
    )(xp, w1p, b1, w2, b2, wd, cb, wb)


def _vtx_prep_body(x_ref, w1_ref, b1_ref, w2_ref, b2_ref, wb_ref, xs_ref, v_ref):
    x = x_ref[...]
    enc = _lrelu(jnp.dot(x, w1_ref[...], preferred_element_type=jnp.float32) + b1_ref[...])
    enc = _lrelu(jnp.dot(enc, w2_ref[...], preferred_element_type=jnp.float32) + b2_ref[...])
    n = enc.shape[0]
    rows = lax.broadcasted_iota(jnp.int32, (n, 1), 0)
    valid = rows < N_VTX
    n2 = jnp.sum(enc * enc, axis=1, keepdims=True)
    n2t = jnp.where(valid, -n2, NEG)
    zer = jnp.zeros((n, AUG - 33), jnp.float32)
    xs_ref[...] = jnp.concatenate([2.0 * enc, n2t, zer], axis=1)
    v = jnp.dot(enc, wb_ref[...], preferred_element_type=jnp.float32)
    v_ref[...] = jnp.concatenate([v, jnp.zeros((n, 96), jnp.float32)], axis=1)


def _vtx_prep(xv, w1p, b1, w2, b2, wb):
    c = lambda a: pl.BlockSpec(a.shape, lambda: (0,) * a.ndim)
    return pl.pallas_call(
        _vtx_prep_body,
        in_specs=[c(xv), c(w1p), c(b1), c(w2), c(b2), c(wb)],
        out_specs=[c(jnp.zeros((NV_PAD, AUG))), c(jnp.zeros((NV_PAD, 128)))],
        out_shape=[jax.ShapeDtypeStruct((NV_PAD, AUG), jnp.float32),
                   jax.ShapeDtypeStruct((NV_PAD, 128), jnp.float32)],
    )(xv, w1p, b1, w2, b2, wb)


def _score_body(xd_ref, xs_ref, s_ref):
    s_ref[...] = lax.dot_general(
        xd_ref[...], xs_ref[...], (((1,), (1,)), ((), ())),
        preferred_element_type=jnp.float32)


def _scores(xd_aug, xs_aug, src_blk):
    nd, ns = xd_aug.shape[0], xs_aug.shape[0]
    dst_blk = 256
    grid = (nd // dst_blk, ns // src_blk)
    return pl.pallas_call(
        _score_body,
        grid=grid,
        in_specs=[pl.BlockSpec((dst_blk, AUG), lambda i, j: (i, 0)),
                  pl.BlockSpec((src_blk, AUG), lambda i, j: (j, 0))],
        out_specs=pl.BlockSpec((dst_blk, src_blk), lambda i, j: (i, j)),
        out_shape=jax.ShapeDtypeStruct((nd, ns), jnp.float32),
    )(xd_aug, xs_aug)


def _make_topk_gather(ncols, rows_per_blk):
    nvregs = ncols // 16
    rpt = NP_PAD // 32
    nblk = rpt // rows_per_blk
    assert nblk % 2 == 0
    nb_gather = rpt // 8
    mesh = plsc.VectorSubcoreMesh(core_axis_name="c", subcore_axis_name="s")

    @functools.partial(
        pl.kernel,
        out_type=jax.ShapeDtypeStruct((NP_PAD * 32,), jnp.float32),
        mesh=mesh,
        compiler_params=pltpu.CompilerParams(needs_layout_passes=False),
        scratch_types=[
            pltpu.VMEM((rows_per_blk, ncols), jnp.float32),
            pltpu.VMEM((rows_per_blk, ncols), jnp.float32),
            pltpu.VMEM((nb_gather, 128), jnp.int32),
            pltpu.VMEM((128, 128), jnp.float32),
            pltpu.VMEM((rpt * 32,), jnp.float32),
            pltpu.SemaphoreType.DMA,
            pltpu.SemaphoreType.DMA,
            pltpu.SemaphoreType.DMA,
        ],
    )
    def topk_kernel(s_hbm, v_hbm, out_hbm, rb0, rb1, idxb, gb, ob, sem0, sem1, gsem):
        cid = lax.axis_index("c")
        sid = lax.axis_index("s")
        wid = sid * 2 + cid
        row_base = wid * rpt
        iota16 = lax.iota(jnp.int32, 16)
        neg16 = jnp.full((16,), NEG, jnp.float32)

        def merge_node(ak, av, bk, bv):
            rbk = lax.rev(bk, (0,))
            rbv = lax.rev(bv, (0,))
            sel = ak >= rbk
            nk = jnp.where(sel, ak, rbk)
            nv = jnp.where(sel, av, rbv)
            sk, sv = plsc.sort_key_val(nk, nv, descending=True)
            return sk, sv

        def top16_tree(pairs):
            while len(pairs) > 1:
                nxt = [merge_node(a[0], a[1], b[0], b[1])
                       for a, b in zip(pairs[0::2], pairs[1::2])]
                if len(pairs) % 2:
                    nxt.append(pairs[-1])
                pairs = nxt
            return pairs[0]

        grp = 16
        n_grp = nvregs // grp

        def scan_row(rbuf, r, row_local):
            def grp_body(g, car):
                tk, tv = car
                base = g * (grp * 16)
                leaves = []
                for t in range(grp):
                    vals = rbuf[r, pl.ds(base + t * 16, 16)]
                    ids = base + t * 16 + iota16
                    leaves.append(plsc.sort_key_val(vals, ids, descending=True))
                sk, sv = top16_tree(leaves)
                return merge_node(tk, tv, sk, sv)

            zero16 = jnp.zeros((16,), jnp.int32)
            tk, tv = lax.fori_loop(0, n_grp, grp_body, (neg16, zero16))
            b_idx = row_local // 8
            lane = (row_local % 8) * 16
            idxb[b_idx, pl.ds(lane, 16)] = tv.astype(jnp.int32)

        def process_blk(rbuf, blk):
            for r in range(rows_per_blk):
                scan_row(rbuf, r, blk * rows_per_blk + r)

        pltpu.async_copy(s_hbm.at[pl.ds(row_base, rows_per_blk)], rb0, sem0)

        def pair_body(p, _):
            b0 = p * 2
            off1 = row_base + (b0 + 1) * rows_per_blk
            pltpu.async_copy(s_hbm.at[pl.ds(off1, rows_per_blk)], rb1, sem1)
            pltpu.make_async_copy(s_hbm.at[pl.ds(row_base, rows_per_blk)], rb0, sem0).wait()
            process_blk(rb0, b0)
            off2 = jnp.minimum(row_base + (b0 + 2) * rows_per_blk,
                               NP_PAD - rows_per_blk)
            pltpu.async_copy(s_hbm.at[pl.ds(off2, rows_per_blk)], rb0, sem0)
            pltpu.make_async_copy(s_hbm.at[pl.ds(off1, rows_per_blk)], rb1, sem1).wait()
            process_blk(rb1, b0 + 1)
            return 0

        lax.fori_loop(0, nblk // 2, pair_body, 0)
        pltpu.make_async_copy(s_hbm.at[pl.ds(row_base, rows_per_blk)], rb0, sem0).wait()

        def gather_body(b, _):
            pltpu.async_copy(v_hbm.at[idxb.at[b]], gb, gsem).wait()
            for rr in range(8):
                m0 = gb[rr * 16, pl.ds(0, 16)]
                m1 = gb[rr * 16, pl.ds(16, 16)]
                for j in range(1, 16):
                    m0 = jnp.maximum(m0, gb[rr * 16 + j, pl.ds(0, 16)])
                    m1 = jnp.maximum(m1, gb[rr * 16 + j, pl.ds(16, 16)])
                row = b * 8 + rr
                ob[pl.ds(row * 32, 16)] = m0
                ob[pl.ds(row * 32 + 16, 16)] = m1
            return 0

        lax.fori_loop(0, nb_gather, gather_body, 0)
        pltpu.sync_copy(ob, out_hbm.at[pl.ds(row_base * 32, rpt * 32)])

    return topk_kernel


_topk_cache = {}


def _topk_gather_1(s, v):
    if 1 not in _topk_cache:
        _topk_cache[1] = _make_topk_gather(NP_PAD, 4)
    return _topk_cache[1](s, v)


def _topk_gather_2(s, v):
    if 2 not in _topk_cache:
        _topk_cache[2] = _make_topk_gather(NV_PAD, 16)
    return _topk_cache[2](s, v)


def _post1_body(u_ref, mv_ref, wd_ref, cb_ref, xd_ref, u2_ref):
    f = _lrelu(u_ref[...] + mv_ref[...])
    n = f.shape[0]
    ones = jnp.ones((n, 1), jnp.float32)
    zer = jnp.zeros((n, AUG - 33), jnp.float32)
    xd_ref[...] = jnp.concatenate([f, ones, zer], axis=1)
    u2_ref[...] = jnp.dot(f, wd_ref[...], preferred_element_type=jnp.float32) + cb_ref[...]


def _post1(u1, mv1, wd, cb):
    blk = 2048
    c = lambda a: pl.BlockSpec(a.shape, lambda i: (0,) * a.ndim)
    return pl.pallas_call(
        _post1_body,
        grid=(NP_PAD // blk,),
        in_specs=[pl.BlockSpec((blk, 32), lambda i: (i, 0)),
                  pl.BlockSpec((blk, 32), lambda i: (i, 0)), c(wd), c(cb)],
        out_specs=[pl.BlockSpec((blk, AUG), lambda i: (i, 0)),
                   pl.BlockSpec((blk, 32), lambda i: (i, 0))],
        out_shape=[jax.ShapeDtypeStruct((NP_PAD, AUG), jnp.float32),
                   jax.ShapeDtypeStruct((NP_PAD, 32), jnp.float32)],
    )(u1, mv1, wd, cb)


def _mlp_body(u_ref, mv_ref, w1_ref, b1_ref, w2_ref, b2_ref, w3_ref, b3_ref,
              w4_ref, b4_ref, o_ref):
    h = _lrelu(u_ref[...] + mv_ref[...])
    h = _lrelu(jnp.dot(h, w1_ref[...], preferred_element_type=jnp.float32) + b1_ref[...])
    h = _lrelu(jnp.dot(h, w2_ref[...], preferred_element_type=jnp.float32) + b2_ref[...])
    h = _lrelu(jnp.dot(h, w3_ref[...], preferred_element_type=jnp.float32) + b3_ref[...])
    h = _lrelu(jnp.dot(h, w4_ref[...], preferred_element_type=jnp.float32) + b4_ref[...])
    o_ref[...] = h


def _out_mlp(u2, mv2, out_w1, out_b1, out_w2, out_b2, out_w3, out_b3, out_w4, out_b4):
    blk = 2048
    w3p = jnp.zeros((32, 128), jnp.float32).at[:, :4].set(out_w3)
    b3p = jnp.zeros((1, 128), jnp.float32).at[0, :4].set(out_b3)
    w4p = jnp.zeros((128, 128), jnp.float32).at[:4, :1].set(out_w4)
    b4p = jnp.zeros((1, 128), jnp.float32).at[0, :1].set(out_b4)
    b1 = out_b1.reshape(1, -1)
    b2 = out_b2.reshape(1, -1)
    c = lambda a: pl.BlockSpec(a.shape, lambda i: (0, 0))
    out = pl.pallas_call(
        _mlp_body,
        grid=(NP_PAD // blk,),
        in_specs=[pl.BlockSpec((blk, 32), lambda i: (i, 0)),
                  pl.BlockSpec((blk, 32), lambda i: (i, 0)),
                  c(out_w1), c(b1), c(out_w2), c(b2),
                  c(w3p), c(b3p), c(w4p), c(b4p)],
        out_specs=pl.BlockSpec((blk, 128), lambda i: (i, 0)),
        out_shape=jax.ShapeDtypeStruct((NP_PAD, 128), jnp.float32),
    )(u2, mv2, out_w1, b1, out_w2, b2, w3p, b3p, w4p, b4p)
    return out[:N_PFC, :1]


def kernel(x_pfc, x_vtx, batch_pfc, batch_vtx, pfc_w1, pfc_b1, pfc_w2, pfc_b2,
           vtx_w1, vtx_b1, vtx_w2, vtx_b2, conv_w, conv_b,
           out_w1, out_b1, out_w2, out_b2, out_w3, out_b3, out_w4, out_b4):
    wb = conv_w[32:]
    wd = conv_w[:32] - wb
    cb = conv_b.reshape(1, -1)

    xp = jnp.zeros((NP_PAD, 8), jnp.float32).at[:N_PFC, :7].set(x_pfc)
    w1p = jnp.zeros((8, 32), jnp.float32).at[:7].set(pfc_w1)
    xd1, xs1, u1, v1 = _pfc_prep(xp, w1p, pfc_b1.reshape(1, -1),
                                 pfc_w2, pfc_b2.reshape(1, -1), wd, cb, wb)

    xv = jnp.zeros((NV_PAD, 8), jnp.float32).at[:N_VTX, :4].set(x_vtx)
    vw1p = jnp.zeros((8, 32), jnp.float32).at[:4].set(vtx_w1)
    xs2, v2 = _vtx_prep(xv, vw1p, vtx_b1.reshape(1, -1), vtx_w2,
                        vtx_b2.reshape(1, -1), wb)

    s1 = _scores(xd1, xs1, 2048)
    mv1 = _topk_gather_1(s1, v1).reshape(NP_PAD, 32)
    xd2, u2 = _post1(u1, mv1, wd, cb)
    s2 = _scores(xd2, xs2, 1024)
    mv2 = _topk_gather_2(s2, v2).reshape(NP_PAD, 32)
    out = _out_mlp(u2, mv2, out_w1, out_b1, out_w2, out_b2,
                   out_w3, out_b3, out_w4, out_b4)
    return (out, batch_pfc)


# speedup vs baseline: 13.1664x; 1.4981x over previous
import functools

import jax
import jax.numpy as jnp
from jax import lax
from jax.experimental import pallas as pl
from jax.experimental.pallas import tpu as pltpu
from jax.experimental.pallas import tpu_sc as plsc

N_PFC = 10000
N_VTX = 1000
NP_PAD = 10240
NV_PAD = 1024
AUG = 40
NEG = -3.0e38


def _lrelu(x):
    return jax.nn.leaky_relu(x, 0.01)


def _pfc_prep_body(x_ref, w1_ref, b1_ref, w2_ref, b2_ref, wd_ref, cb_ref, wb_ref,
                   xd_ref, xs_ref, u_ref, v_ref):
    i = pl.program_id(0)
    x = x_ref[...]
    enc = _lrelu(jnp.dot(x, w1_ref[...], preferred_element_type=jnp.float32) + b1_ref[...])
    enc = _lrelu(jnp.dot(enc, w2_ref[...], preferred_element_type=jnp.float32) + b2_ref[...])
    n = enc.shape[0]
    rows = i * n + lax.broadcasted_iota(jnp.int32, (n, 1), 0)
    valid = rows < N_PFC
    n2 = jnp.sum(enc * enc, axis=1, keepdims=True)
    n2t = jnp.where(valid, -n2, NEG)
    ones = jnp.ones((n, 1), jnp.float32)
    zer = jnp.zeros((n, AUG - 33), jnp.float32)
    xd_ref[...] = jnp.concatenate([enc, ones, zer], axis=1)
    xs_ref[...] = jnp.concatenate([2.0 * enc, n2t, zer], axis=1)
    u_ref[...] = jnp.dot(enc, wd_ref[...], preferred_element_type=jnp.float32) + cb_ref[...]
    v = jnp.dot(enc, wb_ref[...], preferred_element_type=jnp.float32)
    v_ref[...] = jnp.concatenate([v, jnp.zeros((n, 96), jnp.float32)], axis=1)


def _pfc_prep(xp, w1p, b1, w2, b2, wd, cb, wb):
    blk = 2048
    grid = (NP_PAD // blk,)
    c = lambda a: pl.BlockSpec(a.shape, lambda i: (0,) * a.ndim)
    return pl.pallas_call(
        _pfc_prep_body,
        grid=grid,
        in_specs=[pl.BlockSpec((blk, 8), lambda i: (i, 0)),
                  c(w1p), c(b1), c(w2), c(b2), c(wd), c(cb), c(wb)],
        out_specs=[pl.BlockSpec((blk, AUG), lambda i: (i, 0)),
                   pl.BlockSpec((blk, AUG), lambda i: (i, 0)),
                   pl.BlockSpec((blk, 32), lambda i: (i, 0)),
                   pl.BlockSpec((blk, 128), lambda i: (i, 0))],
        out_shape=[jax.ShapeDtypeStruct((NP_PAD, AUG), jnp.float32),
                   jax.ShapeDtypeStruct((NP_PAD, AUG), jnp.float32),
                   jax.ShapeDtypeStruct((NP_PAD, 32), jnp.float32),
                   jax.ShapeDtypeStruct((NP_PAD, 128), jnp.float32)],
    )(xp, w1p, b1, w2, b2, wd, cb, wb)


def _vtx_prep_body(x_ref, w1_ref, b1_ref, w2_ref, b2_ref, wb_ref, xs_ref, v_ref):
    x = x_ref[...]
    enc = _lrelu(jnp.dot(x, w1_ref[...], preferred_element_type=jnp.float32) + b1_ref[...])
    enc = _lrelu(jnp.dot(enc, w2_ref[...], preferred_element_type=jnp.float32) + b2_ref[...])
    n = enc.shape[0]
    rows = lax.broadcasted_iota(jnp.int32, (n, 1), 0)
    valid = rows < N_VTX
    n2 = jnp.sum(enc * enc, axis=1, keepdims=True)
    n2t = jnp.where(valid, -n2, NEG)
    zer = jnp.zeros((n, AUG - 33), jnp.float32)
    xs_ref[...] = jnp.concatenate([2.0 * enc, n2t, zer], axis=1)
    v = jnp.dot(enc, wb_ref[...], preferred_element_type=jnp.float32)
    v_ref[...] = jnp.concatenate([v, jnp.zeros((n, 96), jnp.float32)], axis=1)


def _vtx_prep(xv, w1p, b1, w2, b2, wb):
    c = lambda a: pl.BlockSpec(a.shape, lambda: (0,) * a.ndim)
    return pl.pallas_call(
        _vtx_prep_body,
        in_specs=[c(xv), c(w1p), c(b1), c(w2), c(b2), c(wb)],
        out_specs=[c(jnp.zeros((NV_PAD, AUG))), c(jnp.zeros((NV_PAD, 128)))],
        out_shape=[jax.ShapeDtypeStruct((NV_PAD, AUG), jnp.float32),
                   jax.ShapeDtypeStruct((NV_PAD, 128), jnp.float32)],
    )(xv, w1p, b1, w2, b2, wb)


def _score_body(xd_ref, xs_ref, s_ref):
    s_ref[...] = lax.dot_general(
        xd_ref[...], xs_ref[...], (((1,), (1,)), ((), ())),
        preferred_element_type=jnp.float32)


def _scores(xd_aug, xs_aug, src_blk):
    nd, ns = xd_aug.shape[0], xs_aug.shape[0]
    dst_blk = 256
    grid = (nd // dst_blk, ns // src_blk)
    return pl.pallas_call(
        _score_body,
        grid=grid,
        in_specs=[pl.BlockSpec((dst_blk, AUG), lambda i, j: (i, 0)),
                  pl.BlockSpec((src_blk, AUG), lambda i, j: (j, 0))],
        out_specs=pl.BlockSpec((dst_blk, src_blk), lambda i, j: (i, j)),
        out_shape=jax.ShapeDtypeStruct((nd, ns), jnp.float32),
    )(xd_aug, xs_aug)


def _score3_body(xd_ref, xs_ref, s3_ref, m_ref):
    n = xd_ref.shape[0]
    xd = xd_ref[...]
    maxima = []
    for c in range(5):
        s = lax.dot_general(
            xd, xs_ref[pl.ds(c * 2048, 2048), :], (((1,), (1,)), ((), ())),
            preferred_element_type=jnp.float32)
        s3 = s.reshape(n, 16, 128)
        s3_ref[:, c * 16:(c + 1) * 16, :] = s3
        maxima.append(jnp.max(s3, axis=2))
    maxima.append(jnp.full((n, 48), NEG, jnp.float32))
    m_ref[...] = jnp.concatenate(maxima, axis=1)


def _scores3(xd_aug, xs_aug):
    nd, ns = xd_aug.shape[0], xs_aug.shape[0]
    dst_blk = 256
    grid = (nd // dst_blk,)
    return pl.pallas_call(
        _score3_body,
        grid=grid,
        in_specs=[pl.BlockSpec((dst_blk, AUG), lambda i: (i, 0)),
                  pl.BlockSpec((ns, AUG), lambda i: (0, 0))],
        out_specs=[pl.BlockSpec((dst_blk, 80, 128), lambda i: (i, 0, 0)),
                   pl.BlockSpec((dst_blk, 128), lambda i: (i, 0))],
        out_shape=[jax.ShapeDtypeStruct((nd, ns // 128, 128), jnp.float32),
                   jax.ShapeDtypeStruct((nd, 128), jnp.float32)],
    )(xd_aug, xs_aug)


def _merge_node(ak, av, bk, bv):
    rbk = lax.rev(bk, (0,))
    rbv = lax.rev(bv, (0,))
    sel = ak >= rbk
    nk = jnp.where(sel, ak, rbk)
    nv = jnp.where(sel, av, rbv)
    sk, sv = plsc.sort_key_val(nk, nv, descending=True)
    return sk, sv


def _top16_tree(pairs):
    while len(pairs) > 1:
        nxt = [_merge_node(a[0], a[1], b[0], b[1])
               for a, b in zip(pairs[0::2], pairs[1::2])]
        if len(pairs) % 2:
            nxt.append(pairs[-1])
        pairs = nxt
    return pairs[0]


def _phase_b(v_hbm, idxb, gb0, gb1, ob, gsem0, gsem1, nb_gather):
    def maxrows(gb, b):
        for rr in range(8):
            m0 = gb[rr * 16, pl.ds(0, 16)]
            m1 = gb[rr * 16, pl.ds(16, 16)]
            for j in range(1, 16):
                m0 = jnp.maximum(m0, gb[rr * 16 + j, pl.ds(0, 16)])
                m1 = jnp.maximum(m1, gb[rr * 16 + j, pl.ds(16, 16)])
            row = b * 8 + rr
            ob[pl.ds(row * 32, 16)] = m0
            ob[pl.ds(row * 32 + 16, 16)] = m1

    pltpu.async_copy(v_hbm.at[idxb.at[0]], gb0, gsem0)

    def pair(p, _):
        b0 = p * 2
        pltpu.async_copy(v_hbm.at[idxb.at[b0 + 1]], gb1, gsem1)
        pltpu.make_async_copy(v_hbm.at[idxb.at[0]], gb0, gsem0).wait()
        maxrows(gb0, b0)
        nxt = jnp.minimum(b0 + 2, nb_gather - 1)
        pltpu.async_copy(v_hbm.at[idxb.at[nxt]], gb0, gsem0)
        pltpu.make_async_copy(v_hbm.at[idxb.at[0]], gb1, gsem1).wait()
        maxrows(gb1, b0 + 1)
        return 0

    lax.fori_loop(0, nb_gather // 2, pair, 0)
    pltpu.make_async_copy(v_hbm.at[idxb.at[0]], gb0, gsem0).wait()


def _make_topk_chunked():
    rpt = NP_PAD // 32
    nb_gather = rpt // 8
    mesh = plsc.VectorSubcoreMesh(core_axis_name="c", subcore_axis_name="s")

    @functools.partial(
        pl.kernel,
        out_type=jax.ShapeDtypeStruct((NP_PAD * 32,), jnp.float32),
        mesh=mesh,
        compiler_params=pltpu.CompilerParams(needs_layout_passes=False),
        scratch_types=[
            pltpu.VMEM((rpt, 128), jnp.float32),
            pltpu.VMEM((16,), jnp.int32),
            pltpu.VMEM((16,), jnp.int32),
            pltpu.VMEM((16, 128), jnp.float32),
            pltpu.VMEM((16, 128), jnp.float32),
            pltpu.VMEM((nb_gather, 128), jnp.int32),
            pltpu.VMEM((128, 128), jnp.float32),
            pltpu.VMEM((128, 128), jnp.float32),
            pltpu.VMEM((rpt * 32,), jnp.float32),
            pltpu.SemaphoreType.DMA,
            pltpu.SemaphoreType.DMA,
            pltpu.SemaphoreType.DMA,
            pltpu.SemaphoreType.DMA,
        ],
    )
    def topk_kernel(s3_hbm, m_hbm, v_hbm, out_hbm,
                    mbuf, cidx0, cidx1, cgb0, cgb1, idxb, gb0, gb1, ob,
                    csem0, csem1, gsem0, gsem1):
        cid = lax.axis_index("c")
        sid = lax.axis_index("s")
        wid = sid * 2 + cid
        row_base = wid * rpt
        iota16 = lax.iota(jnp.int32, 16)
        neg16 = jnp.full((16,), NEG, jnp.float32)
        zero16 = jnp.zeros((16,), jnp.int32)

        pltpu.sync_copy(m_hbm.at[pl.ds(row_base, rpt)], mbuf)

        def chunk_sel(r_local, cidx, cgb, csem):
            leaves = [plsc.sort_key_val(mbuf[r_local, pl.ds(c * 16, 16)],
                                        c * 16 + iota16, descending=True)
                      for c in range(5)]
            cmk, cmv = _top16_tree(leaves)
            gidx = (row_base + r_local) * 80 + cmv
            cidx[pl.ds(0, 16)] = gidx
            pltpu.async_copy(s3_hbm.at[cidx], cgb, csem)

        def process_row(r_local, cidx, cgb, csem):
            pltpu.make_async_copy(s3_hbm.at[cidx], cgb, csem).wait()

            def grp_body(g, car):
                tk, tv = car
                leaves = []
                for u in range(16):
                    vals = cgb[g * 2 + (u // 8), pl.ds((u % 8) * 16, 16)]
                    ids = g * 256 + u * 16 + iota16
                    leaves.append(plsc.sort_key_val(vals, ids, descending=True))
                sk, sv = _top16_tree(leaves)
                return _merge_node(tk, tv, sk, sv)

            tk, tv = lax.fori_loop(0, 8, grp_body, (neg16, zero16))
            slot = lax.shift_right_logical(tv, 7)
            g2 = plsc.load_gather(cidx, [slot])
            col = (g2 - (row_base + r_local) * 80) * 128 + (tv & 127)
            b_idx = r_local // 8
            lane = (r_local % 8) * 16
            idxb[b_idx, pl.ds(lane, 16)] = col

        chunk_sel(0, cidx0, cgb0, csem0)

        def pair_body(p, _):
            r0 = p * 2
            chunk_sel(r0 + 1, cidx1, cgb1, csem1)
            process_row(r0, cidx0, cgb0, csem0)
            chunk_sel(jnp.minimum(r0 + 2, rpt - 1), cidx0, cgb0, csem0)
            process_row(r0 + 1, cidx1, cgb1, csem1)
            return 0

        lax.fori_loop(0, rpt // 2, pair_body, 0)
        pltpu.make_async_copy(s3_hbm.at[cidx0], cgb0, csem0).wait()

        _phase_b(v_hbm, idxb, gb0, gb1, ob, gsem0, gsem1, nb_gather)
        pltpu.sync_copy(ob, out_hbm.at[pl.ds(row_base * 32, rpt * 32)])

    return topk_kernel


def _make_topk_gather(ncols, rows_per_blk):
    nvregs = ncols // 16
    rpt = NP_PAD // 32
    nblk = rpt // rows_per_blk
    assert nblk % 2 == 0
    nb_gather = rpt // 8
    mesh = plsc.VectorSubcoreMesh(core_axis_name="c", subcore_axis_name="s")

    @functools.partial(
        pl.kernel,
        out_type=jax.ShapeDtypeStruct((NP_PAD * 32,), jnp.float32),
        mesh=mesh,
        compiler_params=pltpu.CompilerParams(needs_layout_passes=False),
        scratch_types=[
            pltpu.VMEM((rows_per_blk, ncols), jnp.float32),
            pltpu.VMEM((rows_per_blk, ncols), jnp.float32),
            pltpu.VMEM((nb_gather, 128), jnp.int32),
            pltpu.VMEM((128, 128), jnp.float32),
            pltpu.VMEM((128, 128), jnp.float32),
            pltpu.VMEM((rpt * 32,), jnp.float32),
            pltpu.SemaphoreType.DMA,
            pltpu.SemaphoreType.DMA,
            pltpu.SemaphoreType.DMA,
            pltpu.SemaphoreType.DMA,
        ],
    )
    def topk_kernel(s_hbm, v_hbm, out_hbm, rb0, rb1, idxb, gb0, gb1, ob,
                    sem0, sem1, gsem0, gsem1):
        cid = lax.axis_index("c")
        sid = lax.axis_index("s")
        wid = sid * 2 + cid
        row_base = wid * rpt
        iota16 = lax.iota(jnp.int32, 16)
        neg16 = jnp.full((16,), NEG, jnp.float32)

        grp = 16
        n_grp = nvregs // grp

        def scan_row(rbuf, r, row_local):
            def grp_body(g, car):
                tk, tv = car
                base = g * (grp * 16)
                leaves = []
                for t in range(grp):
                    vals = rbuf[r, pl.ds(base + t * 16, 16)]
                    ids = base + t * 16 + iota16
                    leaves.append(plsc.sort_key_val(vals, ids, descending=True))
                sk, sv = _top16_tree(leaves)
                return _merge_node(tk, tv, sk, sv)

            zero16 = jnp.zeros((16,), jnp.int32)
            tk, tv = lax.fori_loop(0, n_grp, grp_body, (neg16, zero16))
            b_idx = row_local // 8
            lane = (row_local % 8) * 16
            idxb[b_idx, pl.ds(lane, 16)] = tv.astype(jnp.int32)

        def process_blk(rbuf, blk):
            for r in range(rows_per_blk):
                scan_row(rbuf, r, blk * rows_per_blk + r)

        pltpu.async_copy(s_hbm.at[pl.ds(row_base, rows_per_blk)], rb0, sem0)

        def pair_body(p, _):
            b0 = p * 2
            off1 = row_base + (b0 + 1) * rows_per_blk
            pltpu.async_copy(s_hbm.at[pl.ds(off1, rows_per_blk)], rb1, sem1)
            pltpu.make_async_copy(s_hbm.at[pl.ds(row_base, rows_per_blk)], rb0, sem0).wait()
            process_blk(rb0, b0)
            off2 = jnp.minimum(row_base + (b0 + 2) * rows_per_blk,
                               NP_PAD - rows_per_blk)
            pltpu.async_copy(s_hbm.at[pl.ds(off2, rows_per_blk)], rb0, sem0)
            pltpu.make_async_copy(s_hbm.at[pl.ds(off1, rows_per_blk)], rb1, sem1).wait()
            process_blk(rb1, b0 + 1)
            return 0

        lax.fori_loop(0, nblk // 2, pair_body, 0)
        pltpu.make_async_copy(s_hbm.at[pl.ds(row_base, rows_per_blk)], rb0, sem0).wait()

        _phase_b(v_hbm, idxb, gb0, gb1, ob, gsem0, gsem1, nb_gather)
        pltpu.sync_copy(ob, out_hbm.at[pl.ds(row_base * 32, rpt * 32)])

    return topk_kernel


_topk_cache = {}


def _topk_gather_1(s3_2d, m, v):
    if 1 not in _topk_cache:
        _topk_cache[1] = _make_topk_chunked()
    return _topk_cache[1](s3_2d, m, v)


def _topk_gather_2(s, v):
    if 2 not in _topk_cache:
        _topk_cache[2] = _make_topk_gather(NV_PAD, 16)
    return _topk_cache[2](s, v)


def _post1_body(u_ref, mv_ref, wd_ref, cb_ref, xd_ref, u2_ref):
    f = _lrelu(u_ref[...] + mv_ref[...])
    n = f.shape[0]
    ones = jnp.ones((n, 1), jnp.float32)
    zer = jnp.zeros((n, AUG - 33), jnp.float32)
    xd_ref[...] = jnp.concatenate([f, ones, zer], axis=1)
    u2_ref[...] = jnp.dot(f, wd_ref[...], preferred_element_type=jnp.float32) + cb_ref[...]


def _post1(u1, mv1, wd, cb):
    blk = 2048
    c = lambda a: pl.BlockSpec(a.shape, lambda i: (0,) * a.ndim)
    return pl.pallas_call(
        _post1_body,
        grid=(NP_PAD // blk,),
        in_specs=[pl.BlockSpec((blk, 32), lambda i: (i, 0)),
                  pl.BlockSpec((blk, 32), lambda i: (i, 0)), c(wd), c(cb)],
        out_specs=[pl.BlockSpec((blk, AUG), lambda i: (i, 0)),
                   pl.BlockSpec((blk, 32), lambda i: (i, 0))],
        out_shape=[jax.ShapeDtypeStruct((NP_PAD, AUG), jnp.float32),
                   jax.ShapeDtypeStruct((NP_PAD, 32), jnp.float32)],
    )(u1, mv1, wd, cb)


def _mlp_body(u_ref, mv_ref, w1_ref, b1_ref, w2_ref, b2_ref, w3_ref, b3_ref,
              w4_ref, b4_ref, o_ref):
    h = _lrelu(u_ref[...] + mv_ref[...])
    h = _lrelu(jnp.dot(h, w1_ref[...], preferred_element_type=jnp.float32) + b1_ref[...])
    h = _lrelu(jnp.dot(h, w2_ref[...], preferred_element_type=jnp.float32) + b2_ref[...])
    h = _lrelu(jnp.dot(h, w3_ref[...], preferred_element_type=jnp.float32) + b3_ref[...])
    h = _lrelu(jnp.dot(h, w4_ref[...], preferred_element_type=jnp.float32) + b4_ref[...])
    o_ref[...] = h


def _out_mlp(u2, mv2, out_w1, out_b1, out_w2, out_b2, out_w3, out_b3, out_w4, out_b4):
    blk = 2048
    w3p = jnp.zeros((32, 128), jnp.float32).at[:, :4].set(out_w3)
    b3p = jnp.zeros((1, 128), jnp.float32).at[0, :4].set(out_b3)
    w4p = jnp.zeros((128, 128), jnp.float32).at[:4, :1].set(out_w4)
    b4p = jnp.zeros((1, 128), jnp.float32).at[0, :1].set(out_b4)
    b1 = out_b1.reshape(1, -1)
    b2 = out_b2.reshape(1, -1)
    c = lambda a: pl.BlockSpec(a.shape, lambda i: (0, 0))
    out = pl.pallas_call(
        _mlp_body,
        grid=(NP_PAD // blk,),
        in_specs=[pl.BlockSpec((blk, 32), lambda i: (i, 0)),
                  pl.BlockSpec((blk, 32), lambda i: (i, 0)),
                  c(out_w1), c(b1), c(out_w2), c(b2),
                  c(w3p), c(b3p), c(w4p), c(b4p)],
        out_specs=pl.BlockSpec((blk, 128), lambda i: (i, 0)),
        out_shape=jax.ShapeDtypeStruct((NP_PAD, 128), jnp.float32),
    )(u2, mv2, out_w1, b1, out_w2, b2, w3p, b3p, w4p, b4p)
    return out[:N_PFC, :1]


def kernel(x_pfc, x_vtx, batch_pfc, batch_vtx, pfc_w1, pfc_b1, pfc_w2, pfc_b2,
           vtx_w1, vtx_b1, vtx_w2, vtx_b2, conv_w, conv_b,
           out_w1, out_b1, out_w2, out_b2, out_w3, out_b3, out_w4, out_b4):
    wb = conv_w[32:]
    wd = conv_w[:32] - wb
    cb = conv_b.reshape(1, -1)

    xp = jnp.zeros((NP_PAD, 8), jnp.float32).at[:N_PFC, :7].set(x_pfc)
    w1p = jnp.zeros((8, 32), jnp.float32).at[:7].set(pfc_w1)
    xd1, xs1, u1, v1 = _pfc_prep(xp, w1p, pfc_b1.reshape(1, -1),
                                 pfc_w2, pfc_b2.reshape(1, -1), wd, cb, wb)

    xv = jnp.zeros((NV_PAD, 8), jnp.float32).at[:N_VTX, :4].set(x_vtx)
    vw1p = jnp.zeros((8, 32), jnp.float32).at[:4].set(vtx_w1)
    xs2, v2 = _vtx_prep(xv, vw1p, vtx_b1.reshape(1, -1), vtx_w2,
                        vtx_b2.reshape(1, -1), wb)

    s3, m1 = _scores3(xd1, xs1)
    s3_2d = s3.reshape(NP_PAD * 80, 128)
    mv1 = _topk_gather_1(s3_2d, m1, v1).reshape(NP_PAD, 32)
    xd2, u2 = _post1(u1, mv1, wd, cb)
    s2 = _scores(xd2, xs2, 1024)
    mv2 = _topk_gather_2(s2, v2).reshape(NP_PAD, 32)
    out = _out_mlp(u2, mv2, out_w1, out_b1, out_w2, out_b2,
                   out_w3, out_b3, out_w4, out_b4)
    return (out, batch_pfc)


# speedup vs baseline: 13.9709x; 1.0611x over previous
import functools

import jax
import jax.numpy as jnp
from jax import lax
from jax.experimental import pallas as pl
from jax.experimental.pallas import tpu as pltpu
from jax.experimental.pallas import tpu_sc as plsc

N_PFC = 10000
N_VTX = 1000
NP_PAD = 10240
NV_PAD = 1024
AUG = 40
NEG = -3.0e38


def _lrelu(x):
    return jax.nn.leaky_relu(x, 0.01)


def _pfc_prep_body(x_ref, w1_ref, b1_ref, w2_ref, b2_ref, wd_ref, cb_ref, wb_ref,
                   xd_ref, xs_ref, u_ref, v_ref):
    i = pl.program_id(0)
    x = x_ref[...]
    enc = _lrelu(jnp.dot(x, w1_ref[...], preferred_element_type=jnp.float32) + b1_ref[...])
    enc = _lrelu(jnp.dot(enc, w2_ref[...], preferred_element_type=jnp.float32) + b2_ref[...])
    n = enc.shape[0]
    rows = i * n + lax.broadcasted_iota(jnp.int32, (n, 1), 0)
    valid = rows < N_PFC
    n2 = jnp.sum(enc * enc, axis=1, keepdims=True)
    n2t = jnp.where(valid, -n2, NEG)
    ones = jnp.ones((n, 1), jnp.float32)
    zer = jnp.zeros((n, AUG - 33), jnp.float32)
    xd_ref[...] = jnp.concatenate([enc, ones, zer], axis=1)
    xs_ref[...] = jnp.concatenate([2.0 * enc, n2t, zer], axis=1)
    u_ref[...] = jnp.dot(enc, wd_ref[...], preferred_element_type=jnp.float32) + cb_ref[...]
    v = jnp.dot(enc, wb_ref[...], preferred_element_type=jnp.float32)
    v_ref[...] = jnp.concatenate([v, jnp.zeros((n, 96), jnp.float32)], axis=1)


def _pfc_prep(xp, w1p, b1, w2, b2, wd, cb, wb):
    blk = 2048
    grid = (NP_PAD // blk,)
    c = lambda a: pl.BlockSpec(a.shape, lambda i: (0,) * a.ndim)
    return pl.pallas_call(
        _pfc_prep_body,
        grid=grid,
        in_specs=[pl.BlockSpec((blk, 8), lambda i: (i, 0)),
                  c(w1p), c(b1), c(w2), c(b2), c(wd), c(cb), c(wb)],
        out_specs=[pl.BlockSpec((blk, AUG), lambda i: (i, 0)),
                   pl.BlockSpec((blk, AUG), lambda i: (i, 0)),
                   pl.BlockSpec((blk, 32), lambda i: (i, 0)),
                   pl.BlockSpec((blk, 128), lambda i: (i, 0))],
        out_shape=[jax.ShapeDtypeStruct((NP_PAD, AUG), jnp.float32),
                   jax.ShapeDtypeStruct((NP_PAD, AUG), jnp.float32),
                   jax.ShapeDtypeStruct((NP_PAD, 32), jnp.float32),
                   jax.ShapeDtypeStruct((NP_PAD, 128), jnp.float32)],
    )(xp, w1p, b1, w2, b2, wd, cb, wb)


def _vtx_prep_body(x_ref, w1_ref, b1_ref, w2_ref, b2_ref, wb_ref, xs_ref, v_ref):
    x = x_ref[...]
    enc = _lrelu(jnp.dot(x, w1_ref[...], preferred_element_type=jnp.float32) + b1_ref[...])
    enc = _lrelu(jnp.dot(enc, w2_ref[...], preferred_element_type=jnp.float32) + b2_ref[...])
    n = enc.shape[0]
    rows = lax.broadcasted_iota(jnp.int32, (n, 1), 0)
    valid = rows < N_VTX
    n2 = jnp.sum(enc * enc, axis=1, keepdims=True)
    n2t = jnp.where(valid, -n2, NEG)
    zer = jnp.zeros((n, AUG - 33), jnp.float32)
    xs_ref[...] = jnp.concatenate([2.0 * enc, n2t, zer], axis=1)
    v = jnp.dot(enc, wb_ref[...], preferred_element_type=jnp.float32)
    v_ref[...] = jnp.concatenate([v, jnp.zeros((n, 96), jnp.float32)], axis=1)


def _vtx_prep(xv, w1p, b1, w2, b2, wb):
    c = lambda a: pl.BlockSpec(a.shape, lambda: (0,) * a.ndim)
    return pl.pallas_call(
        _vtx_prep_body,
        in_specs=[c(xv), c(w1p), c(b1), c(w2), c(b2), c(wb)],
        out_specs=[c(jnp.zeros((NV_PAD, AUG))), c(jnp.zeros((NV_PAD, 128)))],
        out_shape=[jax.ShapeDtypeStruct((NV_PAD, AUG), jnp.float32),
                   jax.ShapeDtypeStruct((NV_PAD, 128), jnp.float32)],
    )(xv, w1p, b1, w2, b2, wb)


def _score_body(xd_ref, xs_ref, s_ref):
    s_ref[...] = lax.dot_general(
        xd_ref[...], xs_ref[...], (((1,), (1,)), ((), ())),
        preferred_element_type=jnp.float32)


def _scores(xd_aug, xs_aug, src_blk):
    nd, ns = xd_aug.shape[0], xs_aug.shape[0]
    dst_blk = 256
    grid = (nd // dst_blk, ns // src_blk)
    return pl.pallas_call(
        _score_body,
        grid=grid,
        in_specs=[pl.BlockSpec((dst_blk, AUG), lambda i, j: (i, 0)),
                  pl.BlockSpec((src_blk, AUG), lambda i, j: (j, 0))],
        out_specs=pl.BlockSpec((dst_blk, src_blk), lambda i, j: (i, j)),
        out_shape=jax.ShapeDtypeStruct((nd, ns), jnp.float32),
    )(xd_aug, xs_aug)


def _score3_body(xd_ref, xs_ref, s3_ref, m_ref):
    n = xd_ref.shape[0]
    xd = xd_ref[...]
    maxima = []
    for c in range(5):
        s = lax.dot_general(
            xd, xs_ref[pl.ds(c * 2048, 2048), :], (((1,), (1,)), ((), ())),
            preferred_element_type=jnp.float32)
        s3 = s.reshape(n, 16, 128)
        s3_ref[:, c * 16:(c + 1) * 16, :] = s3
        maxima.append(jnp.max(s3, axis=2))
    maxima.append(jnp.full((n, 48), NEG, jnp.float32))
    m_ref[...] = jnp.concatenate(maxima, axis=1)


def _scores3(xd_aug, xs_aug):
    nd, ns = xd_aug.shape[0], xs_aug.shape[0]
    dst_blk = 256
    grid = (nd // dst_blk,)
    return pl.pallas_call(
        _score3_body,
        grid=grid,
        in_specs=[pl.BlockSpec((dst_blk, AUG), lambda i: (i, 0)),
                  pl.BlockSpec((ns, AUG), lambda i: (0, 0))],
        out_specs=[pl.BlockSpec((dst_blk, 80, 128), lambda i: (i, 0, 0)),
                   pl.BlockSpec((dst_blk, 128), lambda i: (i, 0))],
        out_shape=[jax.ShapeDtypeStruct((nd, ns // 128, 128), jnp.float32),
                   jax.ShapeDtypeStruct((nd, 128), jnp.float32)],
    )(xd_aug, xs_aug)


def _merge_node(ak, av, bk, bv):
    rbk = lax.rev(bk, (0,))
    rbv = lax.rev(bv, (0,))
    sel = ak >= rbk
    nk = jnp.where(sel, ak, rbk)
    nv = jnp.where(sel, av, rbv)
    sk, sv = plsc.sort_key_val(nk, nv, descending=True)
    return sk, sv


def _top16_tree(pairs):
    while len(pairs) > 1:
        nxt = [_merge_node(a[0], a[1], b[0], b[1])
               for a, b in zip(pairs[0::2], pairs[1::2])]
        if len(pairs) % 2:
            nxt.append(pairs[-1])
        pairs = nxt
    return pairs[0]


def _phase_b(v_hbm, idxb, gb0, gb1, ob, gsem0, gsem1, nb_gather):
    def maxrows(gb, b):
        for rr in range(8):
            m0 = gb[rr * 16, pl.ds(0, 16)]
            m1 = gb[rr * 16, pl.ds(16, 16)]
            for j in range(1, 16):
                m0 = jnp.maximum(m0, gb[rr * 16 + j, pl.ds(0, 16)])
                m1 = jnp.maximum(m1, gb[rr * 16 + j, pl.ds(16, 16)])
            row = b * 8 + rr
            ob[pl.ds(row * 32, 16)] = m0
            ob[pl.ds(row * 32 + 16, 16)] = m1

    pltpu.async_copy(v_hbm.at[idxb.at[0]], gb0, gsem0)

    def pair(p, _):
        b0 = p * 2
        pltpu.async_copy(v_hbm.at[idxb.at[b0 + 1]], gb1, gsem1)
        pltpu.make_async_copy(v_hbm.at[idxb.at[0]], gb0, gsem0).wait()
        maxrows(gb0, b0)
        nxt = jnp.minimum(b0 + 2, nb_gather - 1)
        pltpu.async_copy(v_hbm.at[idxb.at[nxt]], gb0, gsem0)
        pltpu.make_async_copy(v_hbm.at[idxb.at[0]], gb1, gsem1).wait()
        maxrows(gb1, b0 + 1)
        return 0

    lax.fori_loop(0, nb_gather // 2, pair, 0)
    pltpu.make_async_copy(v_hbm.at[idxb.at[0]], gb0, gsem0).wait()


def _make_topk_chunked():
    rpt = NP_PAD // 32
    nb_gather = rpt // 8
    mesh = plsc.VectorSubcoreMesh(core_axis_name="c", subcore_axis_name="s")

    ring = 8

    @functools.partial(
        pl.kernel,
        out_type=jax.ShapeDtypeStruct((NP_PAD * 32,), jnp.float32),
        mesh=mesh,
        compiler_params=pltpu.CompilerParams(needs_layout_passes=False),
        scratch_types=[
            pltpu.VMEM((rpt, 128), jnp.float32),
            pltpu.VMEM((rpt * 16,), jnp.int32),
            pltpu.VMEM((ring, 16, 128), jnp.float32),
            pltpu.VMEM((nb_gather, 128), jnp.int32),
            pltpu.VMEM((128, 128), jnp.float32),
            pltpu.VMEM((128, 128), jnp.float32),
            pltpu.VMEM((rpt * 32,), jnp.float32),
            pltpu.SemaphoreType.DMA((ring,)),
            pltpu.SemaphoreType.DMA,
            pltpu.SemaphoreType.DMA,
        ],
    )
    def topk_kernel(s3_hbm, m_hbm, v_hbm, out_hbm,
                    mbuf, cidxb, cgb, idxb, gb0, gb1, ob,
                    csem, gsem0, gsem1):
        cid = lax.axis_index("c")
        sid = lax.axis_index("s")
        wid = sid * 2 + cid
        row_base = wid * rpt
        iota16 = lax.iota(jnp.int32, 16)
        neg16 = jnp.full((16,), NEG, jnp.float32)
        zero16 = jnp.zeros((16,), jnp.int32)

        pltpu.sync_copy(m_hbm.at[pl.ds(row_base, rpt)], mbuf)

        def chunk_sel(r_local, _):
            leaves = [plsc.sort_key_val(mbuf[r_local, pl.ds(c * 16, 16)],
                                        c * 16 + iota16, descending=True)
                      for c in range(5)]
            cmk, cmv = _top16_tree(leaves)
            cidxb[pl.ds(r_local * 16, 16)] = (row_base + r_local) * 80 + cmv
            return 0

        lax.fori_loop(0, rpt, chunk_sel, 0)

        def issue(r_local, q):
            pltpu.async_copy(s3_hbm.at[cidxb.at[pl.ds(r_local * 16, 16)]],
                             cgb.at[q], csem.at[q])

        def wait(r_local, q):
            pltpu.make_async_copy(s3_hbm.at[cidxb.at[pl.ds(r_local * 16, 16)]],
                                  cgb.at[q], csem.at[q]).wait()

        for q in range(ring):
            issue(q, q)

        def row_body(r, _):
            q = lax.rem(r, ring)
            wait(r, q)

            def grp_body(g, car):
                tk, tv = car
                leaves = []
                for u in range(16):
                    vals = cgb[q, g * 2 + (u // 8), pl.ds((u % 8) * 16, 16)]
                    ids = g * 256 + u * 16 + iota16
                    leaves.append(plsc.sort_key_val(vals, ids, descending=True))
                sk, sv = _top16_tree(leaves)
                return _merge_node(tk, tv, sk, sv)

            tk, tv = lax.fori_loop(0, 8, grp_body, (neg16, zero16))
            slot = lax.shift_right_logical(tv, 7)
            g2 = plsc.load_gather(cidxb, [r * 16 + slot])
            col = (g2 - (row_base + r) * 80) * 128 + (tv & 127)
            b_idx = r // 8
            lane = (r % 8) * 16
            idxb[b_idx, pl.ds(lane, 16)] = col
            issue(jnp.minimum(r + ring, rpt - 1), q)
            return 0

        lax.fori_loop(0, rpt, row_body, 0)
        for q in range(ring):
            wait(0, q)

        _phase_b(v_hbm, idxb, gb0, gb1, ob, gsem0, gsem1, nb_gather)
        pltpu.sync_copy(ob, out_hbm.at[pl.ds(row_base * 32, rpt * 32)])

    return topk_kernel


def _make_topk_gather(ncols, rows_per_blk):
    nvregs = ncols // 16
    rpt = NP_PAD // 32
    nblk = rpt // rows_per_blk
    assert nblk % 2 == 0
    nb_gather = rpt // 8
    mesh = plsc.VectorSubcoreMesh(core_axis_name="c", subcore_axis_name="s")

    @functools.partial(
        pl.kernel,
        out_type=jax.ShapeDtypeStruct((NP_PAD * 32,), jnp.float32),
        mesh=mesh,
        compiler_params=pltpu.CompilerParams(needs_layout_passes=False),
        scratch_types=[
            pltpu.VMEM((rows_per_blk, ncols), jnp.float32),
            pltpu.VMEM((rows_per_blk, ncols), jnp.float32),
            pltpu.VMEM((nb_gather, 128), jnp.int32),
            pltpu.VMEM((128, 128), jnp.float32),
            pltpu.VMEM((128, 128), jnp.float32),
            pltpu.VMEM((rpt * 32,), jnp.float32),
            pltpu.SemaphoreType.DMA,
            pltpu.SemaphoreType.DMA,
            pltpu.SemaphoreType.DMA,
            pltpu.SemaphoreType.DMA,
        ],
    )
    def topk_kernel(s_hbm, v_hbm, out_hbm, rb0, rb1, idxb, gb0, gb1, ob,
                    sem0, sem1, gsem0, gsem1):
        cid = lax.axis_index("c")
        sid = lax.axis_index("s")
        wid = sid * 2 + cid
        row_base = wid * rpt
        iota16 = lax.iota(jnp.int32, 16)
        neg16 = jnp.full((16,), NEG, jnp.float32)

        grp = 16
        n_grp = nvregs // grp

        def scan_row(rbuf, r, row_local):
            def grp_body(g, car):
                tk, tv = car
                base = g * (grp * 16)
                leaves = []
                for t in range(grp):
                    vals = rbuf[r, pl.ds(base + t * 16, 16)]
                    ids = base + t * 16 + iota16
                    leaves.append(plsc.sort_key_val(vals, ids, descending=True))
                sk, sv = _top16_tree(leaves)
                return _merge_node(tk, tv, sk, sv)

            zero16 = jnp.zeros((16,), jnp.int32)
            tk, tv = lax.fori_loop(0, n_grp, grp_body, (neg16, zero16))
            b_idx = row_local // 8
            lane = (row_local % 8) * 16
            idxb[b_idx, pl.ds(lane, 16)] = tv.astype(jnp.int32)

        def process_blk(rbuf, blk):
            for r in range(rows_per_blk):
                scan_row(rbuf, r, blk * rows_per_blk + r)

        pltpu.async_copy(s_hbm.at[pl.ds(row_base, rows_per_blk)], rb0, sem0)

        def pair_body(p, _):
            b0 = p * 2
            off1 = row_base + (b0 + 1) * rows_per_blk
            pltpu.async_copy(s_hbm.at[pl.ds(off1, rows_per_blk)], rb1, sem1)
            pltpu.make_async_copy(s_hbm.at[pl.ds(row_base, rows_per_blk)], rb0, sem0).wait()
            process_blk(rb0, b0)
            off2 = jnp.minimum(row_base + (b0 + 2) * rows_per_blk,
                               NP_PAD - rows_per_blk)
            pltpu.async_copy(s_hbm.at[pl.ds(off2, rows_per_blk)], rb0, sem0)
            pltpu.make_async_copy(s_hbm.at[pl.ds(off1, rows_per_blk)], rb1, sem1).wait()
            process_blk(rb1, b0 + 1)
            return 0

        lax.fori_loop(0, nblk // 2, pair_body, 0)
        pltpu.make_async_copy(s_hbm.at[pl.ds(row_base, rows_per_blk)], rb0, sem0).wait()

        _phase_b(v_hbm, idxb, gb0, gb1, ob, gsem0, gsem1, nb_gather)
        pltpu.sync_copy(ob, out_hbm.at[pl.ds(row_base * 32, rpt * 32)])

    return topk_kernel


_topk_cache = {}


def _topk_gather_1(s3_2d, m, v):
    if 1 not in _topk_cache:
        _topk_cache[1] = _make_topk_chunked()
    return _topk_cache[1](s3_2d, m, v)


def _topk_gather_2(s, v):
    if 2 not in _topk_cache:
        _topk_cache[2] = _make_topk_gather(NV_PAD, 16)
    return _topk_cache[2](s, v)


def _post1_body(u_ref, mv_ref, wd_ref, cb_ref, xd_ref, u2_ref):
    f = _lrelu(u_ref[...] + mv_ref[...])
    n = f.shape[0]
    ones = jnp.ones((n, 1), jnp.float32)
    zer = jnp.zeros((n, AUG - 33), jnp.float32)
    xd_ref[...] = jnp.concatenate([f, ones, zer], axis=1)
    u2_ref[...] = jnp.dot(f, wd_ref[...], preferred_element_type=jnp.float32) + cb_ref[...]


def _post1(u1, mv1, wd, cb):
    blk = 2048
    c = lambda a: pl.BlockSpec(a.shape, lambda i: (0,) * a.ndim)
    return pl.pallas_call(
        _post1_body,
        grid=(NP_PAD // blk,),
        in_specs=[pl.BlockSpec((blk, 32), lambda i: (i, 0)),
                  pl.BlockSpec((blk, 32), lambda i: (i, 0)), c(wd), c(cb)],
        out_specs=[pl.BlockSpec((blk, AUG), lambda i: (i, 0)),
                   pl.BlockSpec((blk, 32), lambda i: (i, 0))],
        out_shape=[jax.ShapeDtypeStruct((NP_PAD, AUG), jnp.float32),
                   jax.ShapeDtypeStruct((NP_PAD, 32), jnp.float32)],
    )(u1, mv1, wd, cb)


def _mlp_body(u_ref, mv_ref, w1_ref, b1_ref, w2_ref, b2_ref, w3_ref, b3_ref,
              w4_ref, b4_ref, o_ref):
    h = _lrelu(u_ref[...] + mv_ref[...])
    h = _lrelu(jnp.dot(h, w1_ref[...], preferred_element_type=jnp.float32) + b1_ref[...])
    h = _lrelu(jnp.dot(h, w2_ref[...], preferred_element_type=jnp.float32) + b2_ref[...])
    h = _lrelu(jnp.dot(h, w3_ref[...], preferred_element_type=jnp.float32) + b3_ref[...])
    h = _lrelu(jnp.dot(h, w4_ref[...], preferred_element_type=jnp.float32) + b4_ref[...])
    o_ref[...] = h


def _out_mlp(u2, mv2, out_w1, out_b1, out_w2, out_b2, out_w3, out_b3, out_w4, out_b4):
    blk = 2048
    w3p = jnp.zeros((32, 128), jnp.float32).at[:, :4].set(out_w3)
    b3p = jnp.zeros((1, 128), jnp.float32).at[0, :4].set(out_b3)
    w4p = jnp.zeros((128, 128), jnp.float32).at[:4, :1].set(out_w4)
    b4p = jnp.zeros((1, 128), jnp.float32).at[0, :1].set(out_b4)
    b1 = out_b1.reshape(1, -1)
    b2 = out_b2.reshape(1, -1)
    c = lambda a: pl.BlockSpec(a.shape, lambda i: (0, 0))
    out = pl.pallas_call(
        _mlp_body,
        grid=(NP_PAD // blk,),
        in_specs=[pl.BlockSpec((blk, 32), lambda i: (i, 0)),
                  pl.BlockSpec((blk, 32), lambda i: (i, 0)),
                  c(out_w1), c(b1), c(out_w2), c(b2),
                  c(w3p), c(b3p), c(w4p), c(b4p)],
        out_specs=pl.BlockSpec((blk, 128), lambda i: (i, 0)),
        out_shape=jax.ShapeDtypeStruct((NP_PAD, 128), jnp.float32),
    )(u2, mv2, out_w1, b1, out_w2, b2, w3p, b3p, w4p, b4p)
    return out[:N_PFC, :1]


def kernel(x_pfc, x_vtx, batch_pfc, batch_vtx, pfc_w1, pfc_b1, pfc_w2, pfc_b2,
           vtx_w1, vtx_b1, vtx_w2, vtx_b2, conv_w, conv_b,
           out_w1, out_b1, out_w2, out_b2, out_w3, out_b3, out_w4, out_b4):
    wb = conv_w[32:]
    wd = conv_w[:32] - wb
    cb = conv_b.reshape(1, -1)

    xp = jnp.zeros((NP_PAD, 8), jnp.float32).at[:N_PFC, :7].set(x_pfc)
    w1p = jnp.zeros((8, 32), jnp.float32).at[:7].set(pfc_w1)
    xd1, xs1, u1, v1 = _pfc_prep(xp, w1p, pfc_b1.reshape(1, -1),
                                 pfc_w2, pfc_b2.reshape(1, -1), wd, cb, wb)

    xv = jnp.zeros((NV_PAD, 8), jnp.float32).at[:N_VTX, :4].set(x_vtx)
    vw1p = jnp.zeros((8, 32), jnp.float32).at[:4].set(vtx_w1)
    xs2, v2 = _vtx_prep(xv, vw1p, vtx_b1.reshape(1, -1), vtx_w2,
                        vtx_b2.reshape(1, -1), wb)

    s3, m1 = _scores3(xd1, xs1)
    s3_2d = s3.reshape(NP_PAD * 80, 128)
    mv1 = _topk_gather_1(s3_2d, m1, v1).reshape(NP_PAD, 32)
    xd2, u2 = _post1(u1, mv1, wd, cb)
    s2 = _scores(xd2, xs2, 1024)
    mv2 = _topk_gather_2(s2, v2).reshape(NP_PAD, 32)
    out = _out_mlp(u2, mv2, out_w1, out_b1, out_w2, out_b2,
                   out_w3, out_b3, out_w4, out_b4)
    return (out, batch_pfc)


# speedup vs baseline: 14.4918x; 1.0373x over previous
import functools

import jax
import jax.numpy as jnp
from jax import lax
from jax.experimental import pallas as pl
from jax.experimental.pallas import tpu as pltpu
from jax.experimental.pallas import tpu_sc as plsc

N_PFC = 10000
N_VTX = 1000
NP_PAD = 10240
NV_PAD = 1024
AUG = 40
NEG = -3.0e38


def _lrelu(x):
    return jax.nn.leaky_relu(x, 0.01)


def _pfc_prep_body(x_ref, w1_ref, b1_ref, w2_ref, b2_ref, wd_ref, cb_ref, wb_ref,
                   xd_ref, xs_ref, u_ref, v_ref):
    i = pl.program_id(0)
    x = x_ref[...]
    enc = _lrelu(jnp.dot(x, w1_ref[...], preferred_element_type=jnp.float32) + b1_ref[...])
    enc = _lrelu(jnp.dot(enc, w2_ref[...], preferred_element_type=jnp.float32) + b2_ref[...])
    n = enc.shape[0]
    rows = i * n + lax.broadcasted_iota(jnp.int32, (n, 1), 0)
    valid = rows < N_PFC
    n2 = jnp.sum(enc * enc, axis=1, keepdims=True)
    n2t = jnp.where(valid, -n2, NEG)
    ones = jnp.ones((n, 1), jnp.float32)
    zer = jnp.zeros((n, AUG - 33), jnp.float32)
    xd_ref[...] = jnp.concatenate([enc, ones, zer], axis=1)
    xs_ref[...] = jnp.concatenate([2.0 * enc, n2t, zer], axis=1)
    u_ref[...] = jnp.dot(enc, wd_ref[...], preferred_element_type=jnp.float32) + cb_ref[...]
    v = jnp.dot(enc, wb_ref[...], preferred_element_type=jnp.float32)
    v_ref[...] = jnp.concatenate([v, jnp.zeros((n, 96), jnp.float32)], axis=1)


def _pfc_prep(xp, w1p, b1, w2, b2, wd, cb, wb):
    blk = 2048
    grid = (NP_PAD // blk,)
    c = lambda a: pl.BlockSpec(a.shape, lambda i: (0,) * a.ndim)
    return pl.pallas_call(
        _pfc_prep_body,
        grid=grid,
        in_specs=[pl.BlockSpec((blk, 8), lambda i: (i, 0)),
                  c(w1p), c(b1), c(w2), c(b2), c(wd), c(cb), c(wb)],
        out_specs=[pl.BlockSpec((blk, AUG), lambda i: (i, 0)),
                   pl.BlockSpec((blk, AUG), lambda i: (i, 0)),
                   pl.BlockSpec((blk, 32), lambda i: (i, 0)),
                   pl.BlockSpec((blk, 128), lambda i: (i, 0))],
        out_shape=[jax.ShapeDtypeStruct((NP_PAD, AUG), jnp.float32),
                   jax.ShapeDtypeStruct((NP_PAD, AUG), jnp.float32),
                   jax.ShapeDtypeStruct((NP_PAD, 32), jnp.float32),
                   jax.ShapeDtypeStruct((NP_PAD, 128), jnp.float32)],
    )(xp, w1p, b1, w2, b2, wd, cb, wb)


def _vtx_prep_body(x_ref, w1_ref, b1_ref, w2_ref, b2_ref, wb_ref, xs_ref, v_ref):
    x = x_ref[...]
    enc = _lrelu(jnp.dot(x, w1_ref[...], preferred_element_type=jnp.float32) + b1_ref[...])
    enc = _lrelu(jnp.dot(enc, w2_ref[...], preferred_element_type=jnp.float32) + b2_ref[...])
    n = enc.shape[0]
    rows = lax.broadcasted_iota(jnp.int32, (n, 1), 0)
    valid = rows < N_VTX
    n2 = jnp.sum(enc * enc, axis=1, keepdims=True)
    n2t = jnp.where(valid, -n2, NEG)
    zer = jnp.zeros((n, AUG - 33), jnp.float32)
    xs_ref[...] = jnp.concatenate([2.0 * enc, n2t, zer], axis=1)
    v = jnp.dot(enc, wb_ref[...], preferred_element_type=jnp.float32)
    v_ref[...] = jnp.concatenate([v, jnp.zeros((n, 96), jnp.float32)], axis=1)


def _vtx_prep(xv, w1p, b1, w2, b2, wb):
    c = lambda a: pl.BlockSpec(a.shape, lambda: (0,) * a.ndim)
    return pl.pallas_call(
        _vtx_prep_body,
        in_specs=[c(xv), c(w1p), c(b1), c(w2), c(b2), c(wb)],
        out_specs=[c(jnp.zeros((NV_PAD, AUG))), c(jnp.zeros((NV_PAD, 128)))],
        out_shape=[jax.ShapeDtypeStruct((NV_PAD, AUG), jnp.float32),
                   jax.ShapeDtypeStruct((NV_PAD, 128), jnp.float32)],
    )(xv, w1p, b1, w2, b2, wb)


def _score_body(xd_ref, xs_ref, s_ref):
    s_ref[...] = lax.dot_general(
        xd_ref[...], xs_ref[...], (((1,), (1,)), ((), ())),
        preferred_element_type=jnp.float32)


def _scores(xd_aug, xs_aug, src_blk):
    nd, ns = xd_aug.shape[0], xs_aug.shape[0]
    dst_blk = 256
    grid = (nd // dst_blk, ns // src_blk)
    return pl.pallas_call(
        _score_body,
        grid=grid,
        in_specs=[pl.BlockSpec((dst_blk, AUG), lambda i, j: (i, 0)),
                  pl.BlockSpec((src_blk, AUG), lambda i, j: (j, 0))],
        out_specs=pl.BlockSpec((dst_blk, src_blk), lambda i, j: (i, j)),
        out_shape=jax.ShapeDtypeStruct((nd, ns), jnp.float32),
    )(xd_aug, xs_aug)


def _score3_body(xd_ref, xs_ref, s3_ref, m_ref):
    n = xd_ref.shape[0]
    xd = xd_ref[...]
    maxima = []
    for c in range(5):
        s = lax.dot_general(
            xd, xs_ref[pl.ds(c * 2048, 2048), :], (((1,), (1,)), ((), ())),
            preferred_element_type=jnp.float32)
        s3 = s.reshape(n, 16, 128)
        s3_ref[:, c * 16:(c + 1) * 16, :] = s3
        maxima.append(jnp.max(s3, axis=2))
    maxima.append(jnp.full((n, 48), NEG, jnp.float32))
    m_ref[...] = jnp.concatenate(maxima, axis=1)


def _scores3(xd_aug, xs_aug):
    nd, ns = xd_aug.shape[0], xs_aug.shape[0]
    dst_blk = 256
    grid = (nd // dst_blk,)
    return pl.pallas_call(
        _score3_body,
        grid=grid,
        in_specs=[pl.BlockSpec((dst_blk, AUG), lambda i: (i, 0)),
                  pl.BlockSpec((ns, AUG), lambda i: (0, 0))],
        out_specs=[pl.BlockSpec((dst_blk, 80, 128), lambda i: (i, 0, 0)),
                   pl.BlockSpec((dst_blk, 128), lambda i: (i, 0))],
        out_shape=[jax.ShapeDtypeStruct((nd, ns // 128, 128), jnp.float32),
                   jax.ShapeDtypeStruct((nd, 128), jnp.float32)],
    )(xd_aug, xs_aug)


def _merge_node(ak, av, bk, bv):
    rbk = lax.rev(bk, (0,))
    rbv = lax.rev(bv, (0,))
    sel = ak >= rbk
    nk = jnp.where(sel, ak, rbk)
    nv = jnp.where(sel, av, rbv)
    sk, sv = plsc.sort_key_val(nk, nv, descending=True)
    return sk, sv


def _top16_tree(pairs):
    while len(pairs) > 1:
        nxt = [_merge_node(a[0], a[1], b[0], b[1])
               for a, b in zip(pairs[0::2], pairs[1::2])]
        if len(pairs) % 2:
            nxt.append(pairs[-1])
        pairs = nxt
    return pairs[0]


def _phase_b(v_hbm, idxb, gb0, gb1, ob, gsem0, gsem1, nb_gather):
    def maxrows(gb, b):
        for rr in range(8):
            m0 = gb[rr * 16, pl.ds(0, 16)]
            m1 = gb[rr * 16, pl.ds(16, 16)]
            for j in range(1, 16):
                m0 = jnp.maximum(m0, gb[rr * 16 + j, pl.ds(0, 16)])
                m1 = jnp.maximum(m1, gb[rr * 16 + j, pl.ds(16, 16)])
            row = b * 8 + rr
            ob[pl.ds(row * 32, 16)] = m0
            ob[pl.ds(row * 32 + 16, 16)] = m1

    pltpu.async_copy(v_hbm.at[idxb.at[0]], gb0, gsem0)

    def pair(p, _):
        b0 = p * 2
        pltpu.async_copy(v_hbm.at[idxb.at[b0 + 1]], gb1, gsem1)
        pltpu.make_async_copy(v_hbm.at[idxb.at[0]], gb0, gsem0).wait()
        maxrows(gb0, b0)
        nxt = jnp.minimum(b0 + 2, nb_gather - 1)
        pltpu.async_copy(v_hbm.at[idxb.at[nxt]], gb0, gsem0)
        pltpu.make_async_copy(v_hbm.at[idxb.at[0]], gb1, gsem1).wait()
        maxrows(gb1, b0 + 1)
        return 0

    lax.fori_loop(0, nb_gather // 2, pair, 0)
    pltpu.make_async_copy(v_hbm.at[idxb.at[0]], gb0, gsem0).wait()


def _make_topk_chunked():
    rpt = NP_PAD // 32
    nb_gather = rpt // 8
    mesh = plsc.VectorSubcoreMesh(core_axis_name="c", subcore_axis_name="s")

    ring = 8

    @functools.partial(
        pl.kernel,
        out_type=jax.ShapeDtypeStruct((NP_PAD * 32,), jnp.float32),
        mesh=mesh,
        compiler_params=pltpu.CompilerParams(needs_layout_passes=False),
        scratch_types=[
            pltpu.VMEM((rpt, 128), jnp.float32),
            pltpu.VMEM((rpt * 16,), jnp.int32),
            pltpu.VMEM((ring, 16, 128), jnp.float32),
            pltpu.VMEM((nb_gather, 128), jnp.int32),
            pltpu.VMEM((128, 128), jnp.float32),
            pltpu.VMEM((128, 128), jnp.float32),
            pltpu.VMEM((rpt * 32,), jnp.float32),
            pltpu.SemaphoreType.DMA((ring,)),
            pltpu.SemaphoreType.DMA,
            pltpu.SemaphoreType.DMA,
        ],
    )
    def topk_kernel(s3_hbm, m_hbm, v_hbm, out_hbm,
                    mbuf, cidxb, cgb, idxb, gb0, gb1, ob,
                    csem, gsem0, gsem1):
        cid = lax.axis_index("c")
        sid = lax.axis_index("s")
        wid = sid * 2 + cid
        row_base = wid * rpt
        iota16 = lax.iota(jnp.int32, 16)
        neg16 = jnp.full((16,), NEG, jnp.float32)
        zero16 = jnp.zeros((16,), jnp.int32)

        pltpu.sync_copy(m_hbm.at[pl.ds(row_base, rpt)], mbuf)

        def chunk_sel(r_local, _):
            leaves = [plsc.sort_key_val(mbuf[r_local, pl.ds(c * 16, 16)],
                                        c * 16 + iota16, descending=True)
                      for c in range(5)]
            cmk, cmv = _top16_tree(leaves)
            cidxb[pl.ds(r_local * 16, 16)] = (row_base + r_local) * 80 + cmv
            return 0

        lax.fori_loop(0, rpt, chunk_sel, 0)

        def issue(r_local, q):
            pltpu.async_copy(s3_hbm.at[cidxb.at[pl.ds(r_local * 16, 16)]],
                             cgb.at[q], csem.at[q])

        def wait(r_local, q):
            pltpu.make_async_copy(s3_hbm.at[cidxb.at[pl.ds(r_local * 16, 16)]],
                                  cgb.at[q], csem.at[q]).wait()

        for q in range(ring):
            issue(q, q)

        def row_body(r, _):
            q = lax.rem(r, ring)
            wait(r, q)

            def grp_body(g, car):
                tk, tv = car
                leaves = []
                for u in range(16):
                    vals = cgb[q, g * 2 + (u // 8), pl.ds((u % 8) * 16, 16)]
                    ids = g * 256 + u * 16 + iota16
                    leaves.append(plsc.sort_key_val(vals, ids, descending=True))
                sk, sv = _top16_tree(leaves)
                return _merge_node(tk, tv, sk, sv)

            tk, tv = lax.fori_loop(0, 8, grp_body, (neg16, zero16))
            slot = lax.shift_right_logical(tv, 7)
            g2 = plsc.load_gather(cidxb, [r * 16 + slot])
            col = (g2 - (row_base + r) * 80) * 128 + (tv & 127)
            b_idx = r // 8
            lane = (r % 8) * 16
            idxb[b_idx, pl.ds(lane, 16)] = col
            issue(jnp.minimum(r + ring, rpt - 1), q)
            return 0

        lax.fori_loop(0, rpt, row_body, 0)
        for q in range(ring):
            wait(0, q)

        _phase_b(v_hbm, idxb, gb0, gb1, ob, gsem0, gsem1, nb_gather)
        pltpu.sync_copy(ob, out_hbm.at[pl.ds(row_base * 32, rpt * 32)])

    return topk_kernel


def _make_topk_gather(ncols, rows_per_blk):
    nvregs = ncols // 16
    rpt = NP_PAD // 32
    nblk = rpt // rows_per_blk
    assert nblk % 2 == 0
    nb_gather = rpt // 8
    mesh = plsc.VectorSubcoreMesh(core_axis_name="c", subcore_axis_name="s")

    @functools.partial(
        pl.kernel,
        out_type=jax.ShapeDtypeStruct((NP_PAD * 32,), jnp.float32),
        mesh=mesh,
        compiler_params=pltpu.CompilerParams(needs_layout_passes=False),
        scratch_types=[
            pltpu.VMEM((rows_per_blk, ncols), jnp.float32),
            pltpu.VMEM((rows_per_blk, ncols), jnp.float32),
            pltpu.VMEM((nb_gather, 128), jnp.int32),
            pltpu.VMEM((128, 128), jnp.float32),
            pltpu.VMEM((128, 128), jnp.float32),
            pltpu.VMEM((rpt * 32,), jnp.float32),
            pltpu.SemaphoreType.DMA,
            pltpu.SemaphoreType.DMA,
            pltpu.SemaphoreType.DMA,
            pltpu.SemaphoreType.DMA,
        ],
    )
    def topk_kernel(s_hbm, v_hbm, out_hbm, rb0, rb1, idxb, gb0, gb1, ob,
                    sem0, sem1, gsem0, gsem1):
        cid = lax.axis_index("c")
        sid = lax.axis_index("s")
        wid = sid * 2 + cid
        row_base = wid * rpt
        iota16 = lax.iota(jnp.int32, 16)
        neg16 = jnp.full((16,), NEG, jnp.float32)

        grp = 16
        n_grp = nvregs // grp

        def scan_row(rbuf, r, row_local):
            def grp_body(g, car):
                tk, tv = car
                base = g * (grp * 16)
                leaves = []
                for t in range(grp):
                    vals = rbuf[r, pl.ds(base + t * 16, 16)]
                    ids = base + t * 16 + iota16
                    leaves.append(plsc.sort_key_val(vals, ids, descending=True))
                sk, sv = _top16_tree(leaves)
                return _merge_node(tk, tv, sk, sv)

            zero16 = jnp.zeros((16,), jnp.int32)
            tk, tv = lax.fori_loop(0, n_grp, grp_body, (neg16, zero16))
            b_idx = row_local // 8
            lane = (row_local % 8) * 16
            idxb[b_idx, pl.ds(lane, 16)] = tv.astype(jnp.int32)

        def process_blk(rbuf, blk):
            def rb(r, _):
                scan_row(rbuf, r, blk * rows_per_blk + r)
                return 0
            lax.fori_loop(0, rows_per_blk, rb, 0)

        pltpu.async_copy(s_hbm.at[pl.ds(row_base, rows_per_blk)], rb0, sem0)

        def pair_body(p, _):
            b0 = p * 2
            off1 = row_base + (b0 + 1) * rows_per_blk
            pltpu.async_copy(s_hbm.at[pl.ds(off1, rows_per_blk)], rb1, sem1)
            pltpu.make_async_copy(s_hbm.at[pl.ds(row_base, rows_per_blk)], rb0, sem0).wait()
            process_blk(rb0, b0)
            off2 = jnp.minimum(row_base + (b0 + 2) * rows_per_blk,
                               NP_PAD - rows_per_blk)
            pltpu.async_copy(s_hbm.at[pl.ds(off2, rows_per_blk)], rb0, sem0)
            pltpu.make_async_copy(s_hbm.at[pl.ds(off1, rows_per_blk)], rb1, sem1).wait()
            process_blk(rb1, b0 + 1)
            return 0

        lax.fori_loop(0, nblk // 2, pair_body, 0)
        pltpu.make_async_copy(s_hbm.at[pl.ds(row_base, rows_per_blk)], rb0, sem0).wait()

        _phase_b(v_hbm, idxb, gb0, gb1, ob, gsem0, gsem1, nb_gather)
        pltpu.sync_copy(ob, out_hbm.at[pl.ds(row_base * 32, rpt * 32)])

    return topk_kernel


_topk_cache = {}


def _topk_gather_1(s3_2d, m, v):
    if 1 not in _topk_cache:
        _topk_cache[1] = _make_topk_chunked()
    return _topk_cache[1](s3_2d, m, v)


def _topk_gather_2(s, v):
    if 2 not in _topk_cache:
        _topk_cache[2] = _make_topk_gather(NV_PAD, 16)
    return _topk_cache[2](s, v)


def _post1_body(u_ref, mv_ref, wd_ref, cb_ref, xd_ref, u2_ref):
    f = _lrelu(u_ref[...] + mv_ref[...])
    n = f.shape[0]
    ones = jnp.ones((n, 1), jnp.float32)
    zer = jnp.zeros((n, AUG - 33), jnp.float32)
    xd_ref[...] = jnp.concatenate([f, ones, zer], axis=1)
    u2_ref[...] = jnp.dot(f, wd_ref[...], preferred_element_type=jnp.float32) + cb_ref[...]


def _post1(u1, mv1, wd, cb):
    blk = 2048
    c = lambda a: pl.BlockSpec(a.shape, lambda i: (0,) * a.ndim)
    return pl.pallas_call(
        _post1_body,
        grid=(NP_PAD // blk,),
        in_specs=[pl.BlockSpec((blk, 32), lambda i: (i, 0)),
                  pl.BlockSpec((blk, 32), lambda i: (i, 0)), c(wd), c(cb)],
        out_specs=[pl.BlockSpec((blk, AUG), lambda i: (i, 0)),
                   pl.BlockSpec((blk, 32), lambda i: (i, 0))],
        out_shape=[jax.ShapeDtypeStruct((NP_PAD, AUG), jnp.float32),
                   jax.ShapeDtypeStruct((NP_PAD, 32), jnp.float32)],
    )(u1, mv1, wd, cb)


def _mlp_body(u_ref, mv_ref, w1_ref, b1_ref, w2_ref, b2_ref, w3_ref, b3_ref,
              w4_ref, b4_ref, o_ref):
    h = _lrelu(u_ref[...] + mv_ref[...])
    h = _lrelu(jnp.dot(h, w1_ref[...], preferred_element_type=jnp.float32) + b1_ref[...])
    h = _lrelu(jnp.dot(h, w2_ref[...], preferred_element_type=jnp.float32) + b2_ref[...])
    h = _lrelu(jnp.dot(h, w3_ref[...], preferred_element_type=jnp.float32) + b3_ref[...])
    h = _lrelu(jnp.dot(h, w4_ref[...], preferred_element_type=jnp.float32) + b4_ref[...])
    o_ref[...] = h


def _out_mlp(u2, mv2, out_w1, out_b1, out_w2, out_b2, out_w3, out_b3, out_w4, out_b4):
    blk = 2048
    w3p = jnp.zeros((32, 128), jnp.float32).at[:, :4].set(out_w3)
    b3p = jnp.zeros((1, 128), jnp.float32).at[0, :4].set(out_b3)
    w4p = jnp.zeros((128, 128), jnp.float32).at[:4, :1].set(out_w4)
    b4p = jnp.zeros((1, 128), jnp.float32).at[0, :1].set(out_b4)
    b1 = out_b1.reshape(1, -1)
    b2 = out_b2.reshape(1, -1)
    c = lambda a: pl.BlockSpec(a.shape, lambda i: (0, 0))
    out = pl.pallas_call(
        _mlp_body,
        grid=(NP_PAD // blk,),
        in_specs=[pl.BlockSpec((blk, 32), lambda i: (i, 0)),
                  pl.BlockSpec((blk, 32), lambda i: (i, 0)),
                  c(out_w1), c(b1), c(out_w2), c(b2),
                  c(w3p), c(b3p), c(w4p), c(b4p)],
        out_specs=pl.BlockSpec((blk, 128), lambda i: (i, 0)),
        out_shape=jax.ShapeDtypeStruct((NP_PAD, 128), jnp.float32),
    )(u2, mv2, out_w1, b1, out_w2, b2, w3p, b3p, w4p, b4p)
    return out[:N_PFC, :1]


def kernel(x_pfc, x_vtx, batch_pfc, batch_vtx, pfc_w1, pfc_b1, pfc_w2, pfc_b2,
           vtx_w1, vtx_b1, vtx_w2, vtx_b2, conv_w, conv_b,
           out_w1, out_b1, out_w2, out_b2, out_w3, out_b3, out_w4, out_b4):
    wb = conv_w[32:]
    wd = conv_w[:32] - wb
    cb = conv_b.reshape(1, -1)

    xp = jnp.zeros((NP_PAD, 8), jnp.float32).at[:N_PFC, :7].set(x_pfc)
    w1p = jnp.zeros((8, 32), jnp.float32).at[:7].set(pfc_w1)
    xd1, xs1, u1, v1 = _pfc_prep(xp, w1p, pfc_b1.reshape(1, -1),
                                 pfc_w2, pfc_b2.reshape(1, -1), wd, cb, wb)

    xv = jnp.zeros((NV_PAD, 8), jnp.float32).at[:N_VTX, :4].set(x_vtx)
    vw1p = jnp.zeros((8, 32), jnp.float32).at[:4].set(vtx_w1)
    xs2, v2 = _vtx_prep(xv, vw1p, vtx_b1.reshape(1, -1), vtx_w2,
                        vtx_b2.reshape(1, -1), wb)

    s3, m1 = _scores3(xd1, xs1)
    s3_2d = s3.reshape(NP_PAD * 80, 128)
    mv1 = _topk_gather_1(s3_2d, m1, v1).reshape(NP_PAD, 32)
    xd2, u2 = _post1(u1, mv1, wd, cb)
    s2 = _scores(xd2, xs2, 1024)
    mv2 = _topk_gather_2(s2, v2).reshape(NP_PAD, 32)
    out = _out_mlp(u2, mv2, out_w1, out_b1, out_w2, out_b2,
                   out_w3, out_b3, out_w4, out_b4)
    return (out, batch_pfc)


# speedup vs baseline: 25.7748x; 1.7786x over previous
import functools

import jax
import jax.numpy as jnp
from jax import lax
from jax.experimental import pallas as pl
from jax.experimental.pallas import tpu as pltpu
from jax.experimental.pallas import tpu_sc as plsc

N_PFC = 10000
N_VTX = 1000
NP_PAD = 10240
NV_PAD = 1024
AUG = 40
NEG = -3.0e38


def _lrelu(x):
    return jax.nn.leaky_relu(x, 0.01)


def _pfc_prep_body(x_ref, w1_ref, b1_ref, w2_ref, b2_ref, wd_ref, cb_ref, wb_ref,
                   xd_ref, xs_ref, u_ref, v_ref):
    i = pl.program_id(0)
    x = x_ref[...]
    enc = _lrelu(jnp.dot(x, w1_ref[...], preferred_element_type=jnp.float32) + b1_ref[...])
    enc = _lrelu(jnp.dot(enc, w2_ref[...], preferred_element_type=jnp.float32) + b2_ref[...])
    n = enc.shape[0]
    rows = i * n + lax.broadcasted_iota(jnp.int32, (n, 1), 0)
    valid = rows < N_PFC
    n2 = jnp.sum(enc * enc, axis=1, keepdims=True)
    n2t = jnp.where(valid, -n2, NEG)
    ones = jnp.ones((n, 1), jnp.float32)
    zer = jnp.zeros((n, AUG - 33), jnp.float32)
    xd_ref[...] = jnp.concatenate([enc, ones, zer], axis=1)
    xs_ref[...] = jnp.concatenate([2.0 * enc, n2t, zer], axis=1)
    u_ref[...] = jnp.dot(enc, wd_ref[...], preferred_element_type=jnp.float32) + cb_ref[...]
    v = jnp.dot(enc, wb_ref[...], preferred_element_type=jnp.float32)
    v_ref[...] = jnp.concatenate([v, jnp.zeros((n, 96), jnp.float32)], axis=1)


def _pfc_prep(xp, w1p, b1, w2, b2, wd, cb, wb):
    blk = 2048
    grid = (NP_PAD // blk,)
    c = lambda a: pl.BlockSpec(a.shape, lambda i: (0,) * a.ndim)
    return pl.pallas_call(
        _pfc_prep_body,
        grid=grid,
        in_specs=[pl.BlockSpec((blk, 8), lambda i: (i, 0)),
                  c(w1p), c(b1), c(w2), c(b2), c(wd), c(cb), c(wb)],
        out_specs=[pl.BlockSpec((blk, AUG), lambda i: (i, 0)),
                   pl.BlockSpec((blk, AUG), lambda i: (i, 0)),
                   pl.BlockSpec((blk, 32), lambda i: (i, 0)),
                   pl.BlockSpec((blk, 128), lambda i: (i, 0))],
        out_shape=[jax.ShapeDtypeStruct((NP_PAD, AUG), jnp.float32),
                   jax.ShapeDtypeStruct((NP_PAD, AUG), jnp.float32),
                   jax.ShapeDtypeStruct((NP_PAD, 32), jnp.float32),
                   jax.ShapeDtypeStruct((NP_PAD, 128), jnp.float32)],
    )(xp, w1p, b1, w2, b2, wd, cb, wb)


def _vtx_prep_body(x_ref, w1_ref, b1_ref, w2_ref, b2_ref, wb_ref, xs_ref, v_ref):
    x = x_ref[...]
    enc = _lrelu(jnp.dot(x, w1_ref[...], preferred_element_type=jnp.float32) + b1_ref[...])
    enc = _lrelu(jnp.dot(enc, w2_ref[...], preferred_element_type=jnp.float32) + b2_ref[...])
    n = enc.shape[0]
    rows = lax.broadcasted_iota(jnp.int32, (n, 1), 0)
    valid = rows < N_VTX
    n2 = jnp.sum(enc * enc, axis=1, keepdims=True)
    n2t = jnp.where(valid, -n2, NEG)
    zer = jnp.zeros((n, AUG - 33), jnp.float32)
    xs_ref[...] = jnp.concatenate([2.0 * enc, n2t, zer], axis=1)
    v_ref[...] = jnp.dot(enc, wb_ref[...], preferred_element_type=jnp.float32)


def _vtx_prep(xv, w1p, b1, w2, b2, wb):
    c = lambda a: pl.BlockSpec(a.shape, lambda: (0,) * a.ndim)
    return pl.pallas_call(
        _vtx_prep_body,
        in_specs=[c(xv), c(w1p), c(b1), c(w2), c(b2), c(wb)],
        out_specs=[c(jnp.zeros((NV_PAD, AUG))), c(jnp.zeros((NV_PAD, 32)))],
        out_shape=[jax.ShapeDtypeStruct((NV_PAD, AUG), jnp.float32),
                   jax.ShapeDtypeStruct((NV_PAD, 32), jnp.float32)],
    )(xv, w1p, b1, w2, b2, wb)


def _score_body(xd_ref, xs_ref, s_ref):
    s_ref[...] = lax.dot_general(
        xd_ref[...], xs_ref[...], (((1,), (1,)), ((), ())),
        preferred_element_type=jnp.float32)


def _scores(xd_aug, xs_aug, src_blk):
    nd, ns = xd_aug.shape[0], xs_aug.shape[0]
    dst_blk = 256
    grid = (nd // dst_blk, ns // src_blk)
    return pl.pallas_call(
        _score_body,
        grid=grid,
        in_specs=[pl.BlockSpec((dst_blk, AUG), lambda i, j: (i, 0)),
                  pl.BlockSpec((src_blk, AUG), lambda i, j: (j, 0))],
        out_specs=pl.BlockSpec((dst_blk, src_blk), lambda i, j: (i, j)),
        out_shape=jax.ShapeDtypeStruct((nd, ns), jnp.float32),
    )(xd_aug, xs_aug)


def _score3_body(xd_ref, xs_ref, s3_ref, m_ref):
    n = xd_ref.shape[0]
    xd = xd_ref[...]
    maxima = []
    for c in range(5):
        s = lax.dot_general(
            xd, xs_ref[pl.ds(c * 2048, 2048), :], (((1,), (1,)), ((), ())),
            preferred_element_type=jnp.float32)
        s3 = s.reshape(n, 16, 128)
        s3_ref[:, c * 16:(c + 1) * 16, :] = s3
        maxima.append(jnp.max(s3, axis=2))
    maxima.append(jnp.full((n, 48), NEG, jnp.float32))
    m_ref[...] = jnp.concatenate(maxima, axis=1)


def _scores3(xd_aug, xs_aug):
    nd, ns = xd_aug.shape[0], xs_aug.shape[0]
    dst_blk = 256
    grid = (nd // dst_blk,)
    return pl.pallas_call(
        _score3_body,
        grid=grid,
        in_specs=[pl.BlockSpec((dst_blk, AUG), lambda i: (i, 0)),
                  pl.BlockSpec((ns, AUG), lambda i: (0, 0))],
        out_specs=[pl.BlockSpec((dst_blk, 80, 128), lambda i: (i, 0, 0)),
                   pl.BlockSpec((dst_blk, 128), lambda i: (i, 0))],
        out_shape=[jax.ShapeDtypeStruct((nd, ns // 128, 128), jnp.float32),
                   jax.ShapeDtypeStruct((nd, 128), jnp.float32)],
    )(xd_aug, xs_aug)


def _merge_node(ak, av, bk, bv):
    rbk = lax.rev(bk, (0,))
    rbv = lax.rev(bv, (0,))
    sel = ak >= rbk
    nk = jnp.where(sel, ak, rbk)
    nv = jnp.where(sel, av, rbv)
    sk, sv = plsc.sort_key_val(nk, nv, descending=True)
    return sk, sv


def _top16_tree(pairs):
    while len(pairs) > 1:
        nxt = [_merge_node(a[0], a[1], b[0], b[1])
               for a, b in zip(pairs[0::2], pairs[1::2])]
        if len(pairs) % 2:
            nxt.append(pairs[-1])
        pairs = nxt
    return pairs[0]


def _phase_b(v_hbm, idxb, gb0, gb1, ob, gsem0, gsem1, nb_gather):
    def maxrows(gb, b):
        for rr in range(8):
            m0 = gb[rr * 16, pl.ds(0, 16)]
            m1 = gb[rr * 16, pl.ds(16, 16)]
            for j in range(1, 16):
                m0 = jnp.maximum(m0, gb[rr * 16 + j, pl.ds(0, 16)])
                m1 = jnp.maximum(m1, gb[rr * 16 + j, pl.ds(16, 16)])
            row = b * 8 + rr
            ob[pl.ds(row * 32, 16)] = m0
            ob[pl.ds(row * 32 + 16, 16)] = m1

    pltpu.async_copy(v_hbm.at[idxb.at[0]], gb0, gsem0)

    def pair(p, _):
        b0 = p * 2
        pltpu.async_copy(v_hbm.at[idxb.at[b0 + 1]], gb1, gsem1)
        pltpu.make_async_copy(v_hbm.at[idxb.at[0]], gb0, gsem0).wait()
        maxrows(gb0, b0)
        nxt = jnp.minimum(b0 + 2, nb_gather - 1)
        pltpu.async_copy(v_hbm.at[idxb.at[nxt]], gb0, gsem0)
        pltpu.make_async_copy(v_hbm.at[idxb.at[0]], gb1, gsem1).wait()
        maxrows(gb1, b0 + 1)
        return 0

    lax.fori_loop(0, nb_gather // 2, pair, 0)
    pltpu.make_async_copy(v_hbm.at[idxb.at[0]], gb0, gsem0).wait()


def _make_topk_chunked():
    rpt = NP_PAD // 32
    nb_gather = rpt // 8
    mesh = plsc.VectorSubcoreMesh(core_axis_name="c", subcore_axis_name="s")

    ring = 8

    @functools.partial(
        pl.kernel,
        out_type=jax.ShapeDtypeStruct((NP_PAD * 32,), jnp.float32),
        mesh=mesh,
        compiler_params=pltpu.CompilerParams(needs_layout_passes=False),
        scratch_types=[
            pltpu.VMEM((rpt, 128), jnp.float32),
            pltpu.VMEM((rpt * 16,), jnp.int32),
            pltpu.VMEM((ring, 16, 128), jnp.float32),
            pltpu.VMEM((nb_gather, 128), jnp.int32),
            pltpu.VMEM((128, 128), jnp.float32),
            pltpu.VMEM((128, 128), jnp.float32),
            pltpu.VMEM((rpt * 32,), jnp.float32),
            pltpu.SemaphoreType.DMA((ring,)),
            pltpu.SemaphoreType.DMA,
            pltpu.SemaphoreType.DMA,
        ],
    )
    def topk_kernel(s3_hbm, m_hbm, v_hbm, out_hbm,
                    mbuf, cidxb, cgb, idxb, gb0, gb1, ob,
                    csem, gsem0, gsem1):
        cid = lax.axis_index("c")
        sid = lax.axis_index("s")
        wid = sid * 2 + cid
        row_base = wid * rpt
        iota16 = lax.iota(jnp.int32, 16)
        neg16 = jnp.full((16,), NEG, jnp.float32)
        zero16 = jnp.zeros((16,), jnp.int32)

        pltpu.sync_copy(m_hbm.at[pl.ds(row_base, rpt)], mbuf)

        def chunk_sel(r_local, _):
            leaves = [plsc.sort_key_val(mbuf[r_local, pl.ds(c * 16, 16)],
                                        c * 16 + iota16, descending=True)
                      for c in range(5)]
            cmk, cmv = _top16_tree(leaves)
            cidxb[pl.ds(r_local * 16, 16)] = (row_base + r_local) * 80 + cmv
            return 0

        lax.fori_loop(0, rpt, chunk_sel, 0)

        def issue(r_local, q):
            pltpu.async_copy(s3_hbm.at[cidxb.at[pl.ds(r_local * 16, 16)]],
                             cgb.at[q], csem.at[q])

        def wait(r_local, q):
            pltpu.make_async_copy(s3_hbm.at[cidxb.at[pl.ds(r_local * 16, 16)]],
                                  cgb.at[q], csem.at[q]).wait()

        for q in range(ring):
            issue(q, q)

        def row_body(r, _):
            q = lax.rem(r, ring)
            wait(r, q)

            def grp_body(g, car):
                tk, tv = car
                leaves = []
                for u in range(16):
                    vals = cgb[q, g * 2 + (u // 8), pl.ds((u % 8) * 16, 16)]
                    ids = g * 256 + u * 16 + iota16
                    leaves.append(plsc.sort_key_val(vals, ids, descending=True))
                sk, sv = _top16_tree(leaves)
                return _merge_node(tk, tv, sk, sv)

            tk, tv = lax.fori_loop(0, 8, grp_body, (neg16, zero16))
            slot = lax.shift_right_logical(tv, 7)
            g2 = plsc.load_gather(cidxb, [r * 16 + slot])
            col = (g2 - (row_base + r) * 80) * 128 + (tv & 127)
            b_idx = r // 8
            lane = (r % 8) * 16
            idxb[b_idx, pl.ds(lane, 16)] = col
            issue(jnp.minimum(r + ring, rpt - 1), q)
            return 0

        lax.fori_loop(0, rpt, row_body, 0)
        for q in range(ring):
            wait(0, q)

        _phase_b(v_hbm, idxb, gb0, gb1, ob, gsem0, gsem1, nb_gather)
        pltpu.sync_copy(ob, out_hbm.at[pl.ds(row_base * 32, rpt * 32)])

    return topk_kernel


def _make_topk_gather(ncols, rows_per_blk):
    nvregs = ncols // 16
    rpt = NP_PAD // 32
    nblk = rpt // rows_per_blk
    assert nblk % 2 == 0
    mesh = plsc.VectorSubcoreMesh(core_axis_name="c", subcore_axis_name="s")

    @functools.partial(
        pl.kernel,
        out_type=jax.ShapeDtypeStruct((NP_PAD * 32,), jnp.float32),
        mesh=mesh,
        compiler_params=pltpu.CompilerParams(needs_layout_passes=False),
        scratch_types=[
            pltpu.VMEM((rows_per_blk, ncols), jnp.float32),
            pltpu.VMEM((rows_per_blk, ncols), jnp.float32),
            pltpu.VMEM((NV_PAD * 32,), jnp.float32),
            pltpu.VMEM((rpt * 32,), jnp.float32),
            pltpu.SemaphoreType.DMA,
            pltpu.SemaphoreType.DMA,
        ],
    )
    def topk_kernel(s_hbm, v_hbm, out_hbm, rb0, rb1, vb, ob, sem0, sem1):
        cid = lax.axis_index("c")
        sid = lax.axis_index("s")
        wid = sid * 2 + cid
        row_base = wid * rpt
        iota16 = lax.iota(jnp.int32, 16)
        neg16 = jnp.full((16,), NEG, jnp.float32)

        pltpu.sync_copy(v_hbm, vb)

        grp = 16
        n_grp = nvregs // grp

        def scan_row(rbuf, r, row_local):
            def grp_body(g, car):
                tk, tv = car
                base = g * (grp * 16)
                leaves = []
                for t in range(grp):
                    vals = rbuf[r, pl.ds(base + t * 16, 16)]
                    ids = base + t * 16 + iota16
                    leaves.append(plsc.sort_key_val(vals, ids, descending=True))
                sk, sv = _top16_tree(leaves)
                return _merge_node(tk, tv, sk, sv)

            zero16 = jnp.zeros((16,), jnp.int32)
            tk, tv = lax.fori_loop(0, n_grp, grp_body, (neg16, zero16))
            off = tv * 32
            j0 = off[0]
            m0 = vb[pl.ds(j0, 16)]
            m1 = vb[pl.ds(j0 + 16, 16)]
            for j in range(1, 16):
                js = off[j]
                m0 = jnp.maximum(m0, vb[pl.ds(js, 16)])
                m1 = jnp.maximum(m1, vb[pl.ds(js + 16, 16)])
            ob[pl.ds(row_local * 32, 16)] = m0
            ob[pl.ds(row_local * 32 + 16, 16)] = m1

        def process_blk(rbuf, blk):
            def rb(r, _):
                scan_row(rbuf, r, blk * rows_per_blk + r)
                return 0
            lax.fori_loop(0, rows_per_blk, rb, 0)

        pltpu.async_copy(s_hbm.at[pl.ds(row_base, rows_per_blk)], rb0, sem0)

        def pair_body(p, _):
            b0 = p * 2
            off1 = row_base + (b0 + 1) * rows_per_blk
            pltpu.async_copy(s_hbm.at[pl.ds(off1, rows_per_blk)], rb1, sem1)
            pltpu.make_async_copy(s_hbm.at[pl.ds(row_base, rows_per_blk)], rb0, sem0).wait()
            process_blk(rb0, b0)
            off2 = jnp.minimum(row_base + (b0 + 2) * rows_per_blk,
                               NP_PAD - rows_per_blk)
            pltpu.async_copy(s_hbm.at[pl.ds(off2, rows_per_blk)], rb0, sem0)
            pltpu.make_async_copy(s_hbm.at[pl.ds(off1, rows_per_blk)], rb1, sem1).wait()
            process_blk(rb1, b0 + 1)
            return 0

        lax.fori_loop(0, nblk // 2, pair_body, 0)
        pltpu.make_async_copy(s_hbm.at[pl.ds(row_base, rows_per_blk)], rb0, sem0).wait()

        pltpu.sync_copy(ob, out_hbm.at[pl.ds(row_base * 32, rpt * 32)])

    return topk_kernel


_topk_cache = {}


def _topk_gather_1(s3_2d, m, v):
    if 1 not in _topk_cache:
        _topk_cache[1] = _make_topk_chunked()
    return _topk_cache[1](s3_2d, m, v)


def _topk_gather_2(s, v):
    if 2 not in _topk_cache:
        _topk_cache[2] = _make_topk_gather(NV_PAD, 16)
    return _topk_cache[2](s, v)


def _post1_body(u_ref, mv_ref, wd_ref, cb_ref, xd_ref, u2_ref):
    f = _lrelu(u_ref[...] + mv_ref[...])
    n = f.shape[0]
    ones = jnp.ones((n, 1), jnp.float32)
    zer = jnp.zeros((n, AUG - 33), jnp.float32)
    xd_ref[...] = jnp.concatenate([f, ones, zer], axis=1)
    u2_ref[...] = jnp.dot(f, wd_ref[...], preferred_element_type=jnp.float32) + cb_ref[...]


def _post1(u1, mv1, wd, cb):
    blk = 2048
    c = lambda a: pl.BlockSpec(a.shape, lambda i: (0,) * a.ndim)
    return pl.pallas_call(
        _post1_body,
        grid=(NP_PAD // blk,),
        in_specs=[pl.BlockSpec((blk, 32), lambda i: (i, 0)),
                  pl.BlockSpec((blk, 32), lambda i: (i, 0)), c(wd), c(cb)],
        out_specs=[pl.BlockSpec((blk, AUG), lambda i: (i, 0)),
                   pl.BlockSpec((blk, 32), lambda i: (i, 0))],
        out_shape=[jax.ShapeDtypeStruct((NP_PAD, AUG), jnp.float32),
                   jax.ShapeDtypeStruct((NP_PAD, 32), jnp.float32)],
    )(u1, mv1, wd, cb)


def _mlp_body(u_ref, mv_ref, w1_ref, b1_ref, w2_ref, b2_ref, w3_ref, b3_ref,
              w4_ref, b4_ref, o_ref):
    h = _lrelu(u_ref[...] + mv_ref[...])
    h = _lrelu(jnp.dot(h, w1_ref[...], preferred_element_type=jnp.float32) + b1_ref[...])
    h = _lrelu(jnp.dot(h, w2_ref[...], preferred_element_type=jnp.float32) + b2_ref[...])
    h = _lrelu(jnp.dot(h, w3_ref[...], preferred_element_type=jnp.float32) + b3_ref[...])
    h = _lrelu(jnp.dot(h, w4_ref[...], preferred_element_type=jnp.float32) + b4_ref[...])
    o_ref[...] = h


def _out_mlp(u2, mv2, out_w1, out_b1, out_w2, out_b2, out_w3, out_b3, out_w4, out_b4):
    blk = 2048
    w3p = jnp.zeros((32, 128), jnp.float32).at[:, :4].set(out_w3)
    b3p = jnp.zeros((1, 128), jnp.float32).at[0, :4].set(out_b3)
    w4p = jnp.zeros((128, 128), jnp.float32).at[:4, :1].set(out_w4)
    b4p = jnp.zeros((1, 128), jnp.float32).at[0, :1].set(out_b4)
    b1 = out_b1.reshape(1, -1)
    b2 = out_b2.reshape(1, -1)
    c = lambda a: pl.BlockSpec(a.shape, lambda i: (0, 0))
    out = pl.pallas_call(
        _mlp_body,
        grid=(NP_PAD // blk,),
        in_specs=[pl.BlockSpec((blk, 32), lambda i: (i, 0)),
                  pl.BlockSpec((blk, 32), lambda i: (i, 0)),
                  c(out_w1), c(b1), c(out_w2), c(b2),
                  c(w3p), c(b3p), c(w4p), c(b4p)],
        out_specs=pl.BlockSpec((blk, 128), lambda i: (i, 0)),
        out_shape=jax.ShapeDtypeStruct((NP_PAD, 128), jnp.float32),
    )(u2, mv2, out_w1, b1, out_w2, b2, w3p, b3p, w4p, b4p)
    return out[:N_PFC, :1]


def kernel(x_pfc, x_vtx, batch_pfc, batch_vtx, pfc_w1, pfc_b1, pfc_w2, pfc_b2,
           vtx_w1, vtx_b1, vtx_w2, vtx_b2, conv_w, conv_b,
           out_w1, out_b1, out_w2, out_b2, out_w3, out_b3, out_w4, out_b4):
    wb = conv_w[32:]
    wd = conv_w[:32] - wb
    cb = conv_b.reshape(1, -1)

    xp = jnp.zeros((NP_PAD, 8), jnp.float32).at[:N_PFC, :7].set(x_pfc)
    w1p = jnp.zeros((8, 32), jnp.float32).at[:7].set(pfc_w1)
    xd1, xs1, u1, v1 = _pfc_prep(xp, w1p, pfc_b1.reshape(1, -1),
                                 pfc_w2, pfc_b2.reshape(1, -1), wd, cb, wb)

    xv = jnp.zeros((NV_PAD, 8), jnp.float32).at[:N_VTX, :4].set(x_vtx)
    vw1p = jnp.zeros((8, 32), jnp.float32).at[:4].set(vtx_w1)
    xs2, v2 = _vtx_prep(xv, vw1p, vtx_b1.reshape(1, -1), vtx_w2,
                        vtx_b2.reshape(1, -1), wb)

    s3, m1 = _scores3(xd1, xs1)
    s3_2d = s3.reshape(NP_PAD * 80, 128)
    mv1 = _topk_gather_1(s3_2d, m1, v1).reshape(NP_PAD, 32)
    xd2, u2 = _post1(u1, mv1, wd, cb)
    s2 = _scores(xd2, xs2, 1024)
    mv2 = _topk_gather_2(s2, v2.reshape(NV_PAD * 32)).reshape(NP_PAD, 32)
    out = _out_mlp(u2, mv2, out_w1, out_b1, out_w2, out_b2,
                   out_w3, out_b3, out_w4, out_b4)
    return (out, batch_pfc)
